# Initial kernel scaffold; baseline (speedup 1.0000x reference)
import functools

import jax
import jax.numpy as jnp
from jax import lax
from jax.experimental import pallas as pl
from jax.experimental.pallas import tpu as pltpu
from jax.experimental.pallas import tpu_sc as plsc

N = 10000
E = 320000
D = 128
NC = 2
NS = 16
NW = NC * NS
EW = E // NW
ET = E // NS
SUB = 80
CB = 400
NSUB = CB // SUB
ROWS2D = E // SUB

_mesh = plsc.VectorSubcoreMesh(core_axis_name="c", subcore_axis_name="s")

f32 = jnp.float32
i32 = jnp.int32


def _wid():
    return lax.axis_index("s") * NC + lax.axis_index("c")


@functools.partial(
    pl.kernel,
    out_type=(
        jax.ShapeDtypeStruct((E, D), f32),
        jax.ShapeDtypeStruct((E,), f32),
    ),
    mesh=_mesh,
    scratch_types=[
        pltpu.VMEM((N,), f32),
        pltpu.VMEM((NSUB, SUB), i32),
        pltpu.VMEM((NSUB, SUB), i32),
        pltpu.VMEM((CB, D), f32),
        pltpu.VMEM((CB,), f32),
        pltpu.SemaphoreType.DMA,
    ],
)
def _sc_gather0(v_hbm, p_hbm, src2_hbm, dst2_hbm, vout, pout,
                ptab, srcv, dstv, rows, pbuf, sem):
    wid = _wid()
    pltpu.sync_copy(p_hbm, ptab)

    def outer(i, carry):
        base = wid * EW + i * CB
        row0 = wid * (EW // SUB) + i * NSUB
        pltpu.sync_copy(src2_hbm.at[pl.ds(row0, NSUB)], srcv)
        pltpu.sync_copy(dst2_hbm.at[pl.ds(row0, NSUB)], dstv)
        descs = []
        for j in range(NSUB):
            descs.append(pltpu.async_copy(
                v_hbm.at[srcv.at[j]], rows.at[pl.ds(j * SUB, SUB)], sem))
        for j in range(NSUB):
            for k in range(SUB // 16):
                idx = dstv[j, pl.ds(k * 16, 16)]
                pbuf[pl.ds(j * SUB + k * 16, 16)] = plsc.load_gather(ptab, [idx])
        for d in descs:
            d.wait()
        pltpu.sync_copy(rows, vout.at[pl.ds(base, CB)])
        pltpu.sync_copy(pbuf, pout.at[pl.ds(base, CB)])
        return carry

    lax.fori_loop(0, EW // CB, outer, 0)


@functools.partial(
    pl.kernel,
    out_type=(
        jax.ShapeDtypeStruct((E,), f32),
        jax.ShapeDtypeStruct((E, D), f32),
    ),
    mesh=_mesh,
    scratch_types=[
        pltpu.VMEM_SHARED((N,), f32),
        pltpu.VMEM((N,), f32),
        pltpu.VMEM((CB,), f32),
        pltpu.VMEM((NSUB, SUB), i32),
        pltpu.VMEM((NSUB, SUB), i32),
        pltpu.VMEM((CB, D), f32),
        pltpu.VMEM((CB,), f32),
        pltpu.SemaphoreType.DMA,
    ],
)
def _sc_seg_gather(w_hbm, src2_hbm, dst2_hbm, tab_hbm, zeros_hbm, ssrc_out, rows_out,
                   S_sh, stile, wv, srcv, dstv, rows, sbuf, sem):
    sid = lax.axis_index("s")
    wid = _wid()

    @pl.when(sid == 0)
    def _():
        pltpu.sync_copy(zeros_hbm, S_sh)

    plsc.subcore_barrier()

    def ph1(i, carry):
        base = sid * ET + i * CB
        row0 = sid * (ET // SUB) + i * NSUB
        pltpu.sync_copy(w_hbm.at[pl.ds(base, CB)], wv)
        pltpu.sync_copy(src2_hbm.at[pl.ds(row0, NSUB)], srcv)
        for j in range(NSUB):
            pltpu.sync_copy(wv.at[pl.ds(j * SUB, SUB)], S_sh.at[srcv.at[j]],
                            add=True)
        return carry

    lax.fori_loop(0, ET // CB, ph1, 0)
    plsc.subcore_barrier()
    pltpu.sync_copy(S_sh, stile)

    def ph2(i, carry):
        base = wid * EW + i * CB
        row0 = wid * (EW // SUB) + i * NSUB
        pltpu.sync_copy(src2_hbm.at[pl.ds(row0, NSUB)], srcv)
        pltpu.sync_copy(dst2_hbm.at[pl.ds(row0, NSUB)], dstv)
        descs = []
        for j in range(NSUB):
            descs.append(pltpu.async_copy(
                tab_hbm.at[dstv.at[j]], rows.at[pl.ds(j * SUB, SUB)], sem))
        for j in range(NSUB):
            for k in range(SUB // 16):
                idx = srcv[j, pl.ds(k * 16, 16)]
                sbuf[pl.ds(j * SUB + k * 16, 16)] = plsc.load_gather(stile, [idx])
        for d in descs:
            d.wait()
        pltpu.sync_copy(rows, rows_out.at[pl.ds(base, CB)])
        pltpu.sync_copy(sbuf, ssrc_out.at[pl.ds(base, CB)])
        return carry

    lax.fori_loop(0, EW // CB, ph2, 0)


@functools.partial(
    pl.kernel,
    out_type=jax.ShapeDtypeStruct((NC, N, D), f32),
    mesh=_mesh,
    scratch_types=[
        pltpu.VMEM_SHARED((N, D), f32),
        pltpu.VMEM((CB, D), f32),
        pltpu.VMEM((NSUB, SUB), i32),
    ],
)
def _sc_scatter(m_hbm, dst2_hbm, zeros_hbm, out,
                X_sh, rows, dstv):
    cid = lax.axis_index("c")
    sid = lax.axis_index("s")
    wid = _wid()

    @pl.when(sid == 0)
    def _():
        pltpu.sync_copy(zeros_hbm, X_sh)

    plsc.subcore_barrier()

    def loop(i, carry):
        base = wid * EW + i * CB
        row0 = wid * (EW // SUB) + i * NSUB
        pltpu.sync_copy(m_hbm.at[pl.ds(base, CB)], rows)
        pltpu.sync_copy(dst2_hbm.at[pl.ds(row0, NSUB)], dstv)
        for j in range(NSUB):
            pltpu.sync_copy(rows.at[pl.ds(j * SUB, SUB)], X_sh.at[dstv.at[j]],
                            add=True)
        return carry

    lax.fori_loop(0, EW // CB, loop, 0)
    plsc.subcore_barrier()

    @pl.when(sid == 0)
    def _():
        pltpu.sync_copy(X_sh, out.at[cid])


@functools.partial(
    pl.kernel,
    out_type=(
        jax.ShapeDtypeStruct((E,), f32),
        jax.ShapeDtypeStruct((E,), f32),
        jax.ShapeDtypeStruct((E, D), f32),
    ),
    mesh=_mesh,
    scratch_types=[
        pltpu.VMEM_SHARED((N,), f32),
        pltpu.VMEM((N,), f32),
        pltpu.VMEM((N,), f32),
        pltpu.VMEM((N,), f32),
        pltpu.VMEM((CB,), f32),
        pltpu.VMEM((NSUB, SUB), i32),
        pltpu.VMEM((NSUB, SUB), i32),
        pltpu.VMEM((CB, D), f32),
        pltpu.VMEM((CB,), f32),
        pltpu.SemaphoreType.DMA,
    ],
)
def _sc_edge1a(p1_hbm, p2_hbm, src2_hbm, dst2_hbm, x1_hbm, zeros_hbm,
               w_out, ssrc_out, rows_out,
               S_sh, p1t, p2t, stile, wv, srcv, dstv, rows, sbuf, sem):
    cid = lax.axis_index("c")
    sid = lax.axis_index("s")
    wid = _wid()
    pltpu.sync_copy(p1_hbm, p1t)
    pltpu.sync_copy(p2_hbm, p2t)

    @pl.when(sid == 0)
    def _():
        pltpu.sync_copy(zeros_hbm, S_sh)

    plsc.subcore_barrier()

    def ph1(i, carry):
        base = sid * ET + i * CB
        row0 = sid * (ET // SUB) + i * NSUB
        pltpu.sync_copy(src2_hbm.at[pl.ds(row0, NSUB)], srcv)
        pltpu.sync_copy(dst2_hbm.at[pl.ds(row0, NSUB)], dstv)
        for j in range(NSUB):
            for k in range(SUB // 16):
                idxd = dstv[j, pl.ds(k * 16, 16)]
                idxs = srcv[j, pl.ds(k * 16, 16)]
                t = plsc.load_gather(p1t, [idxd]) + plsc.load_gather(p2t, [idxs])
                t = jnp.where(t >= 0.0, t, 0.01 * t)
                wv[pl.ds(j * SUB + k * 16, 16)] = jnp.exp(t)
        for j in range(NSUB):
            pltpu.sync_copy(wv.at[pl.ds(j * SUB, SUB)], S_sh.at[srcv.at[j]],
                            add=True)

        @pl.when(cid == 0)
        def _():
            pltpu.sync_copy(wv, w_out.at[pl.ds(base, CB)])

        return carry

    lax.fori_loop(0, ET // CB, ph1, 0)
    plsc.subcore_barrier()
    pltpu.sync_copy(S_sh, stile)

    def ph2(i, carry):
        base = wid * EW + i * CB
        row0 = wid * (EW // SUB) + i * NSUB
        pltpu.sync_copy(src2_hbm.at[pl.ds(row0, NSUB)], srcv)
        pltpu.sync_copy(dst2_hbm.at[pl.ds(row0, NSUB)], dstv)
        descs = []
        for j in range(NSUB):
            descs.append(pltpu.async_copy(
                x1_hbm.at[dstv.at[j]], rows.at[pl.ds(j * SUB, SUB)], sem))
        for j in range(NSUB):
            for k in range(SUB // 16):
                idx = srcv[j, pl.ds(k * 16, 16)]
                sbuf[pl.ds(j * SUB + k * 16, 16)] = plsc.load_gather(stile, [idx])
        for d in descs:
            d.wait()
        pltpu.sync_copy(rows, rows_out.at[pl.ds(base, CB)])
        pltpu.sync_copy(sbuf, ssrc_out.at[pl.ds(base, CB)])
        return carry

    lax.fori_loop(0, EW // CB, ph2, 0)


def _leaky(t):
    return jnp.where(t >= 0.0, t, 0.01 * t)


def _dot(a, b):
    return jnp.dot(a, b, preferred_element_type=f32)


def _tc_prep0_body(x_ref, wa_ref, ba_ref, wn_ref, wal_ref, u_ref, v_ref, p_ref):
    x = x_ref[...]
    u = _leaky(_dot(x, wa_ref[...]) + ba_ref[...])
    u_ref[...] = u
    v_ref[...] = _dot(x, wn_ref[...])
    p_ref[...] = _dot(u, wal_ref[...])


def _tc_prep0(x, W_atom0, b_atom0, Wnb1, Wal0a):
    return pl.pallas_call(
        _tc_prep0_body,
        out_shape=(
            jax.ShapeDtypeStruct((N, D), f32),
            jax.ShapeDtypeStruct((N, D), f32),
            jax.ShapeDtypeStruct((N, 1), f32),
        ),
    )(x, W_atom0, b_atom0.reshape(1, D), Wnb1, Wal0a)


BE = 2560


def _tc_edge0a_body(vs_ref, ea_ref, pd_ref, wn2_ref, bn_ref, wb_ref, bal_ref,
                    w_ref):
    xj = _leaky(vs_ref[...] + _dot(ea_ref[...], wn2_ref[...]) + bn_ref[...])
    ej = _dot(xj, wb_ref[...])
    e = _leaky(pd_ref[...] + ej + bal_ref[...])
    w_ref[...] = jnp.exp(e)


def _tc_edge0a(Vsrc, edge_attr, Pdst, Wnb2, b_nb0, Wb0, b_align0):
    full = lambda s: pl.BlockSpec(s, lambda i: (0, 0))
    return pl.pallas_call(
        _tc_edge0a_body,
        grid=(E // BE,),
        in_specs=[
            pl.BlockSpec((BE, D), lambda i: (i, 0)),
            pl.BlockSpec((BE, 16), lambda i: (i, 0)),
            pl.BlockSpec((BE, 1), lambda i: (i, 0)),
            full((16, D)),
            full((1, D)),
            full((D, 1)),
            full((1, 1)),
        ],
        out_specs=pl.BlockSpec((BE, 1), lambda i: (i, 0)),
        out_shape=jax.ShapeDtypeStruct((E, 1), f32),
    )(Vsrc, edge_attr, Pdst, Wnb2, b_nb0.reshape(1, D), Wb0,
      b_align0.reshape(1, 1))


def _tc_edge_msg_body(xi_ref, w_ref, s_ref, watt_ref, batt_ref,
                      wih_ref, bih_ref, whh_ref, bhh_ref, m_ref):
    x_i = xi_ref[...]
    a = w_ref[...] / (s_ref[...] + 1e-16)
    q = _dot(x_i, watt_ref[...]) + batt_ref[...]
    aq = a * q
    c = jnp.where(aq > 0.0, aq, jnp.expm1(aq))
    gi = _dot(c, wih_ref[...]) + bih_ref[...]
    gh = _dot(x_i, whh_ref[...]) + bhh_ref[...]
    r = jax.nn.sigmoid(gi[:, :D] + gh[:, :D])
    z = jax.nn.sigmoid(gi[:, D:2 * D] + gh[:, D:2 * D])
    n = jnp.tanh(gi[:, 2 * D:] + r * gh[:, 2 * D:])
    m_ref[...] = (1.0 - z) * n + z * x_i


def _tc_edge_msg(Xd, w, Ssrc, W_att, b_att, WihT, bih, WhhT, bhh):
    full = lambda s: pl.BlockSpec(s, lambda i: (0, 0))
    return pl.pallas_call(
        _tc_edge_msg_body,
        grid=(E // BE,),
        in_specs=[
            pl.BlockSpec((BE, D), lambda i: (i, 0)),
            pl.BlockSpec((BE, 1), lambda i: (i, 0)),
            pl.BlockSpec((BE, 1), lambda i: (i, 0)),
            full((D, D)),
            full((1, D)),
            full((D, 3 * D)),
            full((1, 3 * D)),
            full((D, 3 * D)),
            full((1, 3 * D)),
        ],
        out_specs=pl.BlockSpec((BE, D), lambda i: (i, 0)),
        out_shape=jax.ShapeDtypeStruct((E, D), f32),
    )(Xd, w, Ssrc, W_att, b_att.reshape(1, D), WihT, bih.reshape(1, 3 * D),
      WhhT, bhh.reshape(1, 3 * D))


def _tc_prep1_body(xp_ref, wa_ref, wb_ref, bal_ref, x1_ref, p1_ref, p2_ref):
    x1 = xp_ref[0] + xp_ref[1]
    x1_ref[...] = x1
    p1_ref[...] = _dot(x1, wa_ref[...]) + bal_ref[...]
    p2_ref[...] = _dot(x1, wb_ref[...])


def _tc_prep1(x1p, Wa1, Wb1, b_align1):
    return pl.pallas_call(
        _tc_prep1_body,
        out_shape=(
            jax.ShapeDtypeStruct((N, D), f32),
            jax.ShapeDtypeStruct((N, 1), f32),
            jax.ShapeDtypeStruct((N, 1), f32),
        ),
    )(x1p, Wa1, Wb1, b_align1.reshape(1, 1))


def _tc_final_body(xp_ref, out_ref):
    out_ref[...] = xp_ref[0] + xp_ref[1]


def _tc_final(x2p):
    return pl.pallas_call(
        _tc_final_body,
        out_shape=jax.ShapeDtypeStruct((N, D), f32),
    )(x2p)


def kernel(x, edge_index, edge_attr,
           W_atom0, b_atom0, W_nb0, b_nb0, W_align0, b_align0, W_att0, b_att0,
           Wih0, Whh0, bih0, bhh0,
           W_align1, b_align1, W_att1, b_att1, Wih1, Whh1, bih1, bhh1):
    src2d = edge_index[0].reshape(ROWS2D, SUB)
    dst2d = edge_index[1].reshape(ROWS2D, SUB)
    zeros_n = jnp.zeros((N,), f32)
    zeros_nd = jnp.zeros((N, D), f32)

    u, v, p = _tc_prep0(x, W_atom0, b_atom0, W_nb0[:D], W_align0[:D])
    Vsrc, Pdst = _sc_gather0(v, p.reshape(N), src2d, dst2d)
    w = _tc_edge0a(Vsrc, edge_attr, Pdst.reshape(E, 1), W_nb0[D:], b_nb0,
                   W_align0[D:], b_align0)
    Ssrc, Udst = _sc_seg_gather(w.reshape(E), src2d, dst2d, u, zeros_n)
    M = _tc_edge_msg(Udst, w, Ssrc.reshape(E, 1), W_att0, b_att0,
                     Wih0.T, bih0, Whh0.T, bhh0)
    x1p = _sc_scatter(M, dst2d, zeros_nd)

    x1, p1, p2 = _tc_prep1(x1p, W_align1[:D], W_align1[D:], b_align1)
    w1, S1src, X1dst = _sc_edge1a(p1.reshape(N), p2.reshape(N), src2d, dst2d,
                                  x1, zeros_n)
    M1 = _tc_edge_msg(X1dst, w1.reshape(E, 1), S1src.reshape(E, 1),
                      W_att1, b_att1, Wih1.T, bih1, Whh1.T, bhh1)
    x2p = _sc_scatter(M1, dst2d, zeros_nd)
    return _tc_final(x2p)


# speedup vs baseline: 5.4748x; 5.4748x over previous
import functools

import jax
import jax.numpy as jnp
from jax import lax
from jax.experimental import pallas as pl
from jax.experimental.pallas import tpu as pltpu
from jax.experimental.pallas import tpu_sc as plsc

N = 10000
E = 320000
D = 128
NC = 2
NS = 16
NW = NC * NS
EW = E // NW
ET = E // NS
SUB = 80
CB = 400
NSUB = CB // SUB

_mesh = plsc.VectorSubcoreMesh(core_axis_name="c", subcore_axis_name="s")

f32 = jnp.float32
i32 = jnp.int32


def _wid():
    return lax.axis_index("s") * NC + lax.axis_index("c")


def _stage_idx(big, two_d):
    for j in range(NSUB):
        for k in range(SUB // 16):
            two_d[j, pl.ds(k * 16, 16)] = big[pl.ds(j * SUB + k * 16, 16)]


@functools.partial(
    pl.kernel,
    out_type=(
        jax.ShapeDtypeStruct((E, D), f32),
        jax.ShapeDtypeStruct((E,), f32),
    ),
    mesh=_mesh,
    compiler_params=pltpu.CompilerParams(needs_layout_passes=False),
    scratch_types=[
        pltpu.VMEM((N,), f32),
        pltpu.VMEM((CB,), i32),
        pltpu.VMEM((CB,), i32),
        pltpu.VMEM((NSUB, SUB), i32),
        pltpu.VMEM((CB, D), f32),
        pltpu.VMEM((CB,), f32),
        pltpu.SemaphoreType.DMA,
    ],
)
def _sc_gather0(v_hbm, p_hbm, src_hbm, dst_hbm, vout, pout,
                ptab, srcbig, dstbig, srcv, rows, pbuf, sem):
    wid = _wid()
    pltpu.sync_copy(p_hbm, ptab)

    def outer(i, carry):
        base = wid * EW + i * CB
        pltpu.sync_copy(src_hbm.at[pl.ds(base, CB)], srcbig)
        pltpu.sync_copy(dst_hbm.at[pl.ds(base, CB)], dstbig)
        _stage_idx(srcbig, srcv)
        descs = []
        for j in range(NSUB):
            descs.append(pltpu.async_copy(
                v_hbm.at[srcv.at[j]], rows.at[pl.ds(j * SUB, SUB)], sem))
        for k in range(CB // 16):
            idx = dstbig[pl.ds(k * 16, 16)]
            pbuf[pl.ds(k * 16, 16)] = plsc.load_gather(ptab, [idx])
        for d in descs:
            d.wait()
        pltpu.sync_copy(rows, vout.at[pl.ds(base, CB)])
        pltpu.sync_copy(pbuf, pout.at[pl.ds(base, CB)])
        return carry

    lax.fori_loop(0, EW // CB, outer, 0)


@functools.partial(
    pl.kernel,
    out_type=(
        jax.ShapeDtypeStruct((E,), f32),
        jax.ShapeDtypeStruct((E, D), f32),
    ),
    mesh=_mesh,
    compiler_params=pltpu.CompilerParams(needs_layout_passes=False),
    scratch_types=[
        pltpu.VMEM_SHARED((N,), f32),
        pltpu.VMEM((N,), f32),
        pltpu.VMEM((CB,), f32),
        pltpu.VMEM((CB,), i32),
        pltpu.VMEM((CB,), i32),
        pltpu.VMEM((NSUB, SUB), i32),
        pltpu.VMEM((NSUB, SUB), i32),
        pltpu.VMEM((CB, D), f32),
        pltpu.VMEM((CB,), f32),
        pltpu.SemaphoreType.DMA,
    ],
)
def _sc_seg_gather(w_hbm, src_hbm, dst_hbm, tab_hbm, zeros_hbm,
                   ssrc_out, rows_out,
                   S_sh, stile, wv, srcbig, dstbig, srcv, dstv, rows, sbuf,
                   sem):
    sid = lax.axis_index("s")
    wid = _wid()

    @pl.when(sid == 0)
    def _():
        pltpu.sync_copy(zeros_hbm, S_sh)

    plsc.subcore_barrier()

    def ph1(i, carry):
        base = sid * ET + i * CB
        pltpu.sync_copy(w_hbm.at[pl.ds(base, CB)], wv)
        pltpu.sync_copy(src_hbm.at[pl.ds(base, CB)], srcbig)
        _stage_idx(srcbig, srcv)
        for j in range(NSUB):
            pltpu.sync_copy(wv.at[pl.ds(j * SUB, SUB)], S_sh.at[srcv.at[j]],
                            add=True)
        return carry

    lax.fori_loop(0, ET // CB, ph1, 0)
    plsc.subcore_barrier()
    pltpu.sync_copy(S_sh, stile)

    def ph2(i, carry):
        base = wid * EW + i * CB
        pltpu.sync_copy(src_hbm.at[pl.ds(base, CB)], srcbig)
        pltpu.sync_copy(dst_hbm.at[pl.ds(base, CB)], dstbig)
        _stage_idx(dstbig, dstv)
        descs = []
        for j in range(NSUB):
            descs.append(pltpu.async_copy(
                tab_hbm.at[dstv.at[j]], rows.at[pl.ds(j * SUB, SUB)], sem))
        for k in range(CB // 16):
            idx = srcbig[pl.ds(k * 16, 16)]
            sbuf[pl.ds(k * 16, 16)] = plsc.load_gather(stile, [idx])
        for d in descs:
            d.wait()
        pltpu.sync_copy(rows, rows_out.at[pl.ds(base, CB)])
        pltpu.sync_copy(sbuf, ssrc_out.at[pl.ds(base, CB)])
        return carry

    lax.fori_loop(0, EW // CB, ph2, 0)


NH = N // 2
NHP = NH + 8


@functools.partial(
    pl.kernel,
    out_type=jax.ShapeDtypeStruct((N, D), f32),
    mesh=_mesh,
    compiler_params=pltpu.CompilerParams(needs_layout_passes=False),
    scratch_types=[
        pltpu.VMEM_SHARED((NHP, D), f32),
        pltpu.VMEM((CB, D), f32),
        pltpu.VMEM((CB,), i32),
        pltpu.VMEM((NSUB, SUB), i32),
    ],
)
def _sc_scatter(m_hbm, dst_hbm, zeros_hbm, out,
                X_sh, rows, dstbig, dstv):
    cid = lax.axis_index("c")
    sid = lax.axis_index("s")
    lo = cid * NH

    @pl.when(sid == 0)
    def _():
        pltpu.sync_copy(zeros_hbm.at[pl.ds(0, NHP)], X_sh)

    plsc.subcore_barrier()

    def loop(i, carry):
        base = sid * ET + i * CB
        pltpu.sync_copy(m_hbm.at[pl.ds(base, CB)], rows)
        pltpu.sync_copy(dst_hbm.at[pl.ds(base, CB)], dstbig)
        for j in range(NSUB):
            for k in range(SUB // 16):
                t = dstbig[pl.ds(j * SUB + k * 16, 16)] - lo
                t = jnp.where((t >= 0) & (t < NH), t, NH)
                dstv[j, pl.ds(k * 16, 16)] = t
        for j in range(NSUB):
            pltpu.sync_copy(rows.at[pl.ds(j * SUB, SUB)], X_sh.at[dstv.at[j]],
                            add=True)
        return carry

    lax.fori_loop(0, ET // CB, loop, 0)
    plsc.subcore_barrier()

    @pl.when(sid == 0)
    def _():
        pltpu.sync_copy(X_sh.at[pl.ds(0, NH)], out.at[pl.ds(lo, NH)])


@functools.partial(
    pl.kernel,
    out_type=(
        jax.ShapeDtypeStruct((E,), f32),
        jax.ShapeDtypeStruct((E,), f32),
        jax.ShapeDtypeStruct((E, D), f32),
    ),
    mesh=_mesh,
    compiler_params=pltpu.CompilerParams(needs_layout_passes=False),
    scratch_types=[
        pltpu.VMEM_SHARED((N,), f32),
        pltpu.VMEM((N,), f32),
        pltpu.VMEM((N,), f32),
        pltpu.VMEM((N,), f32),
        pltpu.VMEM((CB,), f32),
        pltpu.VMEM((CB,), i32),
        pltpu.VMEM((CB,), i32),
        pltpu.VMEM((NSUB, SUB), i32),
        pltpu.VMEM((NSUB, SUB), i32),
        pltpu.VMEM((CB, D), f32),
        pltpu.VMEM((CB,), f32),
        pltpu.SemaphoreType.DMA,
    ],
)
def _sc_edge1a(p1_hbm, p2_hbm, src_hbm, dst_hbm, x1_hbm, zeros_hbm,
               w_out, ssrc_out, rows_out,
               S_sh, p1t, p2t, stile, wv, srcbig, dstbig, srcv, dstv, rows,
               sbuf, sem):
    cid = lax.axis_index("c")
    sid = lax.axis_index("s")
    wid = _wid()
    pltpu.sync_copy(p1_hbm, p1t)
    pltpu.sync_copy(p2_hbm, p2t)

    @pl.when(sid == 0)
    def _():
        pltpu.sync_copy(zeros_hbm, S_sh)

    plsc.subcore_barrier()

    def ph1(i, carry):
        base = sid * ET + i * CB
        pltpu.sync_copy(src_hbm.at[pl.ds(base, CB)], srcbig)
        pltpu.sync_copy(dst_hbm.at[pl.ds(base, CB)], dstbig)
        _stage_idx(srcbig, srcv)
        for k in range(CB // 16):
            idxd = dstbig[pl.ds(k * 16, 16)]
            idxs = srcbig[pl.ds(k * 16, 16)]
            t = plsc.load_gather(p1t, [idxd]) + plsc.load_gather(p2t, [idxs])
            t = jnp.where(t >= 0.0, t, 0.01 * t)
            wv[pl.ds(k * 16, 16)] = jnp.exp(t)
        for j in range(NSUB):
            pltpu.sync_copy(wv.at[pl.ds(j * SUB, SUB)], S_sh.at[srcv.at[j]],
                            add=True)

        @pl.when(cid == 0)
        def _():
            pltpu.sync_copy(wv, w_out.at[pl.ds(base, CB)])

        return carry

    lax.fori_loop(0, ET // CB, ph1, 0)
    plsc.subcore_barrier()
    pltpu.sync_copy(S_sh, stile)

    def ph2(i, carry):
        base = wid * EW + i * CB
        pltpu.sync_copy(src_hbm.at[pl.ds(base, CB)], srcbig)
        pltpu.sync_copy(dst_hbm.at[pl.ds(base, CB)], dstbig)
        _stage_idx(dstbig, dstv)
        descs = []
        for j in range(NSUB):
            descs.append(pltpu.async_copy(
                x1_hbm.at[dstv.at[j]], rows.at[pl.ds(j * SUB, SUB)], sem))
        for k in range(CB // 16):
            idx = srcbig[pl.ds(k * 16, 16)]
            sbuf[pl.ds(k * 16, 16)] = plsc.load_gather(stile, [idx])
        for d in descs:
            d.wait()
        pltpu.sync_copy(rows, rows_out.at[pl.ds(base, CB)])
        pltpu.sync_copy(sbuf, ssrc_out.at[pl.ds(base, CB)])
        return carry

    lax.fori_loop(0, EW // CB, ph2, 0)


def _leaky(t):
    return jnp.where(t >= 0.0, t, 0.01 * t)


def _dot(a, b):
    return jnp.dot(a, b, preferred_element_type=f32)


def _tc_prep0_body(x_ref, wa_ref, ba_ref, wn_ref, wal_ref, u_ref, v_ref, p_ref):
    x = x_ref[...]
    u = _leaky(_dot(x, wa_ref[...]) + ba_ref[...])
    u_ref[...] = u
    v_ref[...] = _dot(x, wn_ref[...])
    p_ref[...] = _dot(u, wal_ref[...])


def _tc_prep0(x, W_atom0, b_atom0, Wnb1, Wal0a):
    return pl.pallas_call(
        _tc_prep0_body,
        out_shape=(
            jax.ShapeDtypeStruct((N, D), f32),
            jax.ShapeDtypeStruct((N, D), f32),
            jax.ShapeDtypeStruct((N, 1), f32),
        ),
    )(x, W_atom0, b_atom0.reshape(1, D), Wnb1, Wal0a)


BE = 2560


def _tc_edge0a_body(vs_ref, ea_ref, pd_ref, wn2_ref, bn_ref, wb_ref, bal_ref,
                    w_ref):
    xj = _leaky(vs_ref[...] + _dot(ea_ref[...], wn2_ref[...]) + bn_ref[...])
    ej = _dot(xj, wb_ref[...])
    e = _leaky(pd_ref[...] + ej + bal_ref[...])
    w_ref[...] = jnp.exp(e)


def _tc_edge0a(Vsrc, edge_attr, Pdst, Wnb2, b_nb0, Wb0, b_align0):
    full = lambda s: pl.BlockSpec(s, lambda i: (0, 0))
    return pl.pallas_call(
        _tc_edge0a_body,
        grid=(E // BE,),
        in_specs=[
            pl.BlockSpec((BE, D), lambda i: (i, 0)),
            pl.BlockSpec((BE, 16), lambda i: (i, 0)),
            pl.BlockSpec((BE, 1), lambda i: (i, 0)),
            full((16, D)),
            full((1, D)),
            full((D, 1)),
            full((1, 1)),
        ],
        out_specs=pl.BlockSpec((BE, 1), lambda i: (i, 0)),
        out_shape=jax.ShapeDtypeStruct((E, 1), f32),
    )(Vsrc, edge_attr, Pdst, Wnb2, b_nb0.reshape(1, D), Wb0,
      b_align0.reshape(1, 1))


def _tc_edge_msg_body(xi_ref, w_ref, s_ref, watt_ref, batt_ref,
                      wih_ref, bih_ref, whh_ref, bhh_ref, m_ref):
    x_i = xi_ref[...]
    a = w_ref[...] / (s_ref[...] + 1e-16)
    q = _dot(x_i, watt_ref[...]) + batt_ref[...]
    aq = a * q
    c = jnp.where(aq > 0.0, aq, jnp.exp(aq) - 1.0)
    gi = _dot(c, wih_ref[...]) + bih_ref[...]
    gh = _dot(x_i, whh_ref[...]) + bhh_ref[...]
    r = jax.nn.sigmoid(gi[:, :D] + gh[:, :D])
    z = jax.nn.sigmoid(gi[:, D:2 * D] + gh[:, D:2 * D])
    n = jnp.tanh(gi[:, 2 * D:] + r * gh[:, 2 * D:])
    m_ref[...] = (1.0 - z) * n + z * x_i


def _tc_edge_msg(Xd, w, Ssrc, W_att, b_att, WihT, bih, WhhT, bhh):
    full = lambda s: pl.BlockSpec(s, lambda i: (0, 0))
    return pl.pallas_call(
        _tc_edge_msg_body,
        grid=(E // BE,),
        in_specs=[
            pl.BlockSpec((BE, D), lambda i: (i, 0)),
            pl.BlockSpec((BE, 1), lambda i: (i, 0)),
            pl.BlockSpec((BE, 1), lambda i: (i, 0)),
            full((D, D)),
            full((1, D)),
            full((D, 3 * D)),
            full((1, 3 * D)),
            full((D, 3 * D)),
            full((1, 3 * D)),
        ],
        out_specs=pl.BlockSpec((BE, D), lambda i: (i, 0)),
        out_shape=jax.ShapeDtypeStruct((E, D), f32),
    )(Xd, w, Ssrc, W_att, b_att.reshape(1, D), WihT, bih.reshape(1, 3 * D),
      WhhT, bhh.reshape(1, 3 * D))


def _tc_prep1_body(x1_ref, wa_ref, wb_ref, bal_ref, p1_ref, p2_ref):
    x1 = x1_ref[...]
    p1_ref[...] = _dot(x1, wa_ref[...]) + bal_ref[...]
    p2_ref[...] = _dot(x1, wb_ref[...])


def _tc_prep1(x1, Wa1, Wb1, b_align1):
    return pl.pallas_call(
        _tc_prep1_body,
        out_shape=(
            jax.ShapeDtypeStruct((N, 1), f32),
            jax.ShapeDtypeStruct((N, 1), f32),
        ),
    )(x1, Wa1, Wb1, b_align1.reshape(1, 1))


def kernel(x, edge_index, edge_attr,
           W_atom0, b_atom0, W_nb0, b_nb0, W_align0, b_align0, W_att0, b_att0,
           Wih0, Whh0, bih0, bhh0,
           W_align1, b_align1, W_att1, b_att1, Wih1, Whh1, bih1, bhh1):
    src = edge_index[0]
    dst = edge_index[1]
    zeros_n = jnp.zeros((N,), f32)
    zeros_nd = jnp.zeros((N, D), f32)

    u, v, p = _tc_prep0(x, W_atom0, b_atom0, W_nb0[:D], W_align0[:D])
    Vsrc, Pdst = _sc_gather0(v, p.reshape(N), src, dst)
    w = _tc_edge0a(Vsrc, edge_attr, Pdst.reshape(E, 1), W_nb0[D:], b_nb0,
                   W_align0[D:], b_align0)
    Ssrc, Udst = _sc_seg_gather(w.reshape(E), src, dst, u, zeros_n)
    M = _tc_edge_msg(Udst, w, Ssrc.reshape(E, 1), W_att0, b_att0,
                     Wih0.T, bih0, Whh0.T, bhh0)
    x1 = _sc_scatter(M, dst, zeros_nd)

    p1, p2 = _tc_prep1(x1, W_align1[:D], W_align1[D:], b_align1)
    w1, S1src, X1dst = _sc_edge1a(p1.reshape(N), p2.reshape(N), src, dst,
                                  x1, zeros_n)
    M1 = _tc_edge_msg(X1dst, w1.reshape(E, 1), S1src.reshape(E, 1),
                      W_att1, b_att1, Wih1.T, bih1, Whh1.T, bhh1)
    return _sc_scatter(M1, dst, zeros_nd)


# speedup vs baseline: 7.0329x; 1.2846x over previous
import functools

import jax
import jax.numpy as jnp
from jax import lax
from jax.experimental import pallas as pl
from jax.experimental.pallas import tpu as pltpu
from jax.experimental.pallas import tpu_sc as plsc

N = 10000
E = 320000
D = 128
NC = 2
NS = 16
NW = NC * NS
EW = E // NW
ET = E // NS
SUB = 80
CB = 400
NSUB = CB // SUB

_mesh = plsc.VectorSubcoreMesh(core_axis_name="c", subcore_axis_name="s")

f32 = jnp.float32
i32 = jnp.int32


def _wid():
    return lax.axis_index("s") * NC + lax.axis_index("c")


def _stage_idx(big, two_d):
    for j in range(NSUB):
        for k in range(SUB // 16):
            two_d[j, pl.ds(k * 16, 16)] = big[pl.ds(j * SUB + k * 16, 16)]


@functools.partial(
    pl.kernel,
    out_type=(
        jax.ShapeDtypeStruct((E, D), f32),
        jax.ShapeDtypeStruct((E,), f32),
    ),
    mesh=_mesh,
    compiler_params=pltpu.CompilerParams(needs_layout_passes=False),
    scratch_types=[
        pltpu.VMEM((N,), f32),
        pltpu.VMEM((CB,), i32),
        pltpu.VMEM((CB,), i32),
        pltpu.VMEM((NSUB, SUB), i32),
        pltpu.VMEM((CB, D), f32),
        pltpu.VMEM((CB,), f32),
        pltpu.SemaphoreType.DMA,
    ],
)
def _sc_gather0(v_hbm, p_hbm, src_hbm, dst_hbm, vout, pout,
                ptab, srcbig, dstbig, srcv, rows, pbuf, sem):
    wid = _wid()
    pltpu.sync_copy(p_hbm, ptab)

    def outer(i, carry):
        base = wid * EW + i * CB
        pltpu.sync_copy(src_hbm.at[pl.ds(base, CB)], srcbig)
        pltpu.sync_copy(dst_hbm.at[pl.ds(base, CB)], dstbig)
        _stage_idx(srcbig, srcv)
        descs = []
        for j in range(NSUB):
            descs.append(pltpu.async_copy(
                v_hbm.at[srcv.at[j]], rows.at[pl.ds(j * SUB, SUB)], sem))
        for k in range(CB // 16):
            idx = dstbig[pl.ds(k * 16, 16)]
            pbuf[pl.ds(k * 16, 16)] = plsc.load_gather(ptab, [idx])
        for d in descs:
            d.wait()
        pltpu.sync_copy(rows, vout.at[pl.ds(base, CB)])
        pltpu.sync_copy(pbuf, pout.at[pl.ds(base, CB)])
        return carry

    lax.fori_loop(0, EW // CB, outer, 0)


@functools.partial(
    pl.kernel,
    out_type=(
        jax.ShapeDtypeStruct((E,), f32),
        jax.ShapeDtypeStruct((E, D), f32),
    ),
    mesh=_mesh,
    compiler_params=pltpu.CompilerParams(needs_layout_passes=False),
    scratch_types=[
        pltpu.VMEM_SHARED((N,), f32),
        pltpu.VMEM((N,), f32),
        pltpu.VMEM((CB,), f32),
        pltpu.VMEM((CB,), i32),
        pltpu.VMEM((CB,), i32),
        pltpu.VMEM((NSUB, SUB), i32),
        pltpu.VMEM((NSUB, SUB), i32),
        pltpu.VMEM((CB, D), f32),
        pltpu.VMEM((CB,), f32),
        pltpu.SemaphoreType.DMA,
    ],
)
def _sc_seg_gather(w_hbm, src_hbm, dst_hbm, tab_hbm, zeros_hbm,
                   ssrc_out, rows_out,
                   S_sh, stile, wv, srcbig, dstbig, srcv, dstv, rows, sbuf,
                   sem):
    sid = lax.axis_index("s")
    wid = _wid()

    @pl.when(sid == 0)
    def _():
        pltpu.sync_copy(zeros_hbm, S_sh)

    plsc.subcore_barrier()

    def ph1(i, carry):
        base = sid * ET + i * CB
        pltpu.sync_copy(w_hbm.at[pl.ds(base, CB)], wv)
        pltpu.sync_copy(src_hbm.at[pl.ds(base, CB)], srcbig)
        _stage_idx(srcbig, srcv)
        for j in range(NSUB):
            pltpu.sync_copy(wv.at[pl.ds(j * SUB, SUB)], S_sh.at[srcv.at[j]],
                            add=True)
        return carry

    lax.fori_loop(0, ET // CB, ph1, 0)
    plsc.subcore_barrier()
    pltpu.sync_copy(S_sh, stile)

    def ph2(i, carry):
        base = wid * EW + i * CB
        pltpu.sync_copy(src_hbm.at[pl.ds(base, CB)], srcbig)
        pltpu.sync_copy(dst_hbm.at[pl.ds(base, CB)], dstbig)
        pltpu.sync_copy(w_hbm.at[pl.ds(base, CB)], wv)
        _stage_idx(dstbig, dstv)
        descs = []
        for j in range(NSUB):
            descs.append(pltpu.async_copy(
                tab_hbm.at[dstv.at[j]], rows.at[pl.ds(j * SUB, SUB)], sem))
        for k in range(CB // 16):
            idx = srcbig[pl.ds(k * 16, 16)]
            sv = plsc.load_gather(stile, [idx])
            sbuf[pl.ds(k * 16, 16)] = wv[pl.ds(k * 16, 16)] / (sv + 1e-16)
        for d in descs:
            d.wait()
        pltpu.sync_copy(rows, rows_out.at[pl.ds(base, CB)])
        pltpu.sync_copy(sbuf, ssrc_out.at[pl.ds(base, CB)])
        return carry

    lax.fori_loop(0, EW // CB, ph2, 0)


NH = N // 2
NHP = NH + 8


@functools.partial(
    pl.kernel,
    out_type=jax.ShapeDtypeStruct((N, D), f32),
    mesh=_mesh,
    compiler_params=pltpu.CompilerParams(needs_layout_passes=False),
    scratch_types=[
        pltpu.VMEM_SHARED((NHP, D), f32),
        pltpu.VMEM((CB, D), f32),
        pltpu.VMEM((CB,), i32),
        pltpu.VMEM((NSUB, SUB), i32),
    ],
)
def _sc_scatter(m_hbm, dst_hbm, zeros_hbm, out,
                X_sh, rows, dstbig, dstv):
    cid = lax.axis_index("c")
    sid = lax.axis_index("s")
    lo = cid * NH

    @pl.when(sid == 0)
    def _():
        pltpu.sync_copy(zeros_hbm.at[pl.ds(0, NHP)], X_sh)

    plsc.subcore_barrier()

    def loop(i, carry):
        base = sid * ET + i * CB
        pltpu.sync_copy(m_hbm.at[pl.ds(base, CB)], rows)
        pltpu.sync_copy(dst_hbm.at[pl.ds(base, CB)], dstbig)
        for j in range(NSUB):
            for k in range(SUB // 16):
                t = dstbig[pl.ds(j * SUB + k * 16, 16)] - lo
                t = jnp.where((t >= 0) & (t < NH), t, NH)
                dstv[j, pl.ds(k * 16, 16)] = t
        for j in range(NSUB):
            pltpu.sync_copy(rows.at[pl.ds(j * SUB, SUB)], X_sh.at[dstv.at[j]],
                            add=True)
        return carry

    lax.fori_loop(0, ET // CB, loop, 0)
    plsc.subcore_barrier()

    @pl.when(sid == 0)
    def _():
        pltpu.sync_copy(X_sh.at[pl.ds(0, NH)], out.at[pl.ds(lo, NH)])


@functools.partial(
    pl.kernel,
    out_type=(
        jax.ShapeDtypeStruct((E,), f32),
        jax.ShapeDtypeStruct((E, D), f32),
    ),
    mesh=_mesh,
    compiler_params=pltpu.CompilerParams(needs_layout_passes=False),
    scratch_types=[
        pltpu.VMEM_SHARED((N,), f32),
        pltpu.VMEM((N,), f32),
        pltpu.VMEM((N,), f32),
        pltpu.VMEM((N,), f32),
        pltpu.VMEM((CB,), f32),
        pltpu.VMEM((CB,), i32),
        pltpu.VMEM((CB,), i32),
        pltpu.VMEM((NSUB, SUB), i32),
        pltpu.VMEM((NSUB, SUB), i32),
        pltpu.VMEM((CB, D), f32),
        pltpu.VMEM((CB,), f32),
        pltpu.SemaphoreType.DMA,
    ],
)
def _sc_edge1a(p1_hbm, p2_hbm, src_hbm, dst_hbm, x1_hbm, zeros_hbm,
               ssrc_out, rows_out,
               S_sh, p1t, p2t, stile, wv, srcbig, dstbig, srcv, dstv, rows,
               sbuf, sem):
    cid = lax.axis_index("c")
    sid = lax.axis_index("s")
    wid = _wid()
    pltpu.sync_copy(p1_hbm, p1t)
    pltpu.sync_copy(p2_hbm, p2t)

    @pl.when(sid == 0)
    def _():
        pltpu.sync_copy(zeros_hbm, S_sh)

    plsc.subcore_barrier()

    def ph1(i, carry):
        base = sid * ET + i * CB
        pltpu.sync_copy(src_hbm.at[pl.ds(base, CB)], srcbig)
        pltpu.sync_copy(dst_hbm.at[pl.ds(base, CB)], dstbig)
        _stage_idx(srcbig, srcv)
        for k in range(CB // 16):
            idxd = dstbig[pl.ds(k * 16, 16)]
            idxs = srcbig[pl.ds(k * 16, 16)]
            t = plsc.load_gather(p1t, [idxd]) + plsc.load_gather(p2t, [idxs])
            t = jnp.where(t >= 0.0, t, 0.01 * t)
            wv[pl.ds(k * 16, 16)] = jnp.exp(t)
        for j in range(NSUB):
            pltpu.sync_copy(wv.at[pl.ds(j * SUB, SUB)], S_sh.at[srcv.at[j]],
                            add=True)
        return carry

    lax.fori_loop(0, ET // CB, ph1, 0)
    plsc.subcore_barrier()
    pltpu.sync_copy(S_sh, stile)

    def ph2(i, carry):
        base = wid * EW + i * CB
        pltpu.sync_copy(src_hbm.at[pl.ds(base, CB)], srcbig)
        pltpu.sync_copy(dst_hbm.at[pl.ds(base, CB)], dstbig)
        _stage_idx(dstbig, dstv)
        descs = []
        for j in range(NSUB):
            descs.append(pltpu.async_copy(
                x1_hbm.at[dstv.at[j]], rows.at[pl.ds(j * SUB, SUB)], sem))
        for k in range(CB // 16):
            idxd = dstbig[pl.ds(k * 16, 16)]
            idxs = srcbig[pl.ds(k * 16, 16)]
            t = plsc.load_gather(p1t, [idxd]) + plsc.load_gather(p2t, [idxs])
            t = jnp.where(t >= 0.0, t, 0.01 * t)
            sv = plsc.load_gather(stile, [idxs])
            sbuf[pl.ds(k * 16, 16)] = jnp.exp(t) / (sv + 1e-16)
        for d in descs:
            d.wait()
        pltpu.sync_copy(rows, rows_out.at[pl.ds(base, CB)])
        pltpu.sync_copy(sbuf, ssrc_out.at[pl.ds(base, CB)])
        return carry

    lax.fori_loop(0, EW // CB, ph2, 0)


def _leaky(t):
    return jnp.where(t >= 0.0, t, 0.01 * t)


def _dot(a, b):
    return jnp.dot(a, b, preferred_element_type=f32)


def _tc_prep0_body(x_ref, wa_ref, ba_ref, wn_ref, wal_ref, u_ref, v_ref, p_ref):
    x = x_ref[...]
    u = _leaky(_dot(x, wa_ref[...]) + ba_ref[...])
    u_ref[...] = u
    v_ref[...] = _dot(x, wn_ref[...])
    p_ref[...] = _dot(u, wal_ref[...])


def _tc_prep0(x, W_atom0, b_atom0, Wnb1, Wal0a):
    return pl.pallas_call(
        _tc_prep0_body,
        out_shape=(
            jax.ShapeDtypeStruct((N, D), f32),
            jax.ShapeDtypeStruct((N, D), f32),
            jax.ShapeDtypeStruct((N, 1), f32),
        ),
    )(x, W_atom0, b_atom0.reshape(1, D), Wnb1, Wal0a)


BE = 2560
BEH = BE // 128


def _tc_edge0a_body(vs_ref, ea_ref, pd_ref, wn2_ref, bn_ref, wb_ref, bal_ref,
                    w_ref):
    ea = lax.dot_general(ea_ref[...], wn2_ref[...], (((0,), (0,)), ((), ())),
                         preferred_element_type=f32)
    xj = _leaky(vs_ref[...] + ea + bn_ref[...])
    ej = _dot(xj, wb_ref[...])
    pdt = jnp.transpose(pd_ref[0])
    pdcol = jnp.concatenate([pdt[:, r:r + 1] for r in range(BEH)], axis=0)
    e = _leaky(pdcol + ej + bal_ref[...])
    w = jnp.exp(e)
    rows = [jnp.transpose(w[r * 128:(r + 1) * 128, :]) for r in range(BEH)]
    w_ref[...] = jnp.reshape(jnp.concatenate(rows, axis=0), (1, BEH, 128))


def _tc_edge0a(Vsrc, eaT, Pdst, Wnb2, b_nb0, Wb0, b_align0):
    full = lambda s: pl.BlockSpec(s, lambda i: (0, 0))
    return pl.pallas_call(
        _tc_edge0a_body,
        grid=(E // BE,),
        in_specs=[
            pl.BlockSpec((BE, D), lambda i: (i, 0)),
            pl.BlockSpec((16, BE), lambda i: (0, i)),
            pl.BlockSpec((1, BEH, 128), lambda i: (i, 0, 0)),
            full((16, D)),
            full((1, D)),
            full((D, 1)),
            full((1, 1)),
        ],
        out_specs=pl.BlockSpec((1, BEH, 128), lambda i: (i, 0, 0)),
        out_shape=jax.ShapeDtypeStruct((E // BE, BEH, 128), f32),
    )(Vsrc, eaT, Pdst, Wnb2, b_nb0.reshape(1, D), Wb0,
      b_align0.reshape(1, 1))


def _tc_edge_msg_body(xi_ref, a_ref, watt_ref, batt_ref,
                      wih_ref, bih_ref, whh_ref, bhh_ref, m_ref):
    x_i = xi_ref[...]
    at = jnp.transpose(a_ref[0])
    q = _dot(x_i, watt_ref[...]) + batt_ref[...]
    aq = jnp.concatenate(
        [q[r * 128:(r + 1) * 128, :] * at[:, r:r + 1] for r in range(BEH)],
        axis=0)
    c = jnp.where(aq > 0.0, aq, jnp.exp(aq) - 1.0)
    gi = _dot(c, wih_ref[...]) + bih_ref[...]
    gh = _dot(x_i, whh_ref[...]) + bhh_ref[...]
    r = jax.nn.sigmoid(gi[:, :D] + gh[:, :D])
    z = jax.nn.sigmoid(gi[:, D:2 * D] + gh[:, D:2 * D])
    n = jnp.tanh(gi[:, 2 * D:] + r * gh[:, 2 * D:])
    m_ref[...] = (1.0 - z) * n + z * x_i


def _tc_edge_msg(Xd, a, W_att, b_att, WihT, bih, WhhT, bhh):
    full = lambda s: pl.BlockSpec(s, lambda i: (0, 0))
    return pl.pallas_call(
        _tc_edge_msg_body,
        grid=(E // BE,),
        in_specs=[
            pl.BlockSpec((BE, D), lambda i: (i, 0)),
            pl.BlockSpec((1, BEH, 128), lambda i: (i, 0, 0)),
            full((D, D)),
            full((1, D)),
            full((D, 3 * D)),
            full((1, 3 * D)),
            full((D, 3 * D)),
            full((1, 3 * D)),
        ],
        out_specs=pl.BlockSpec((BE, D), lambda i: (i, 0)),
        out_shape=jax.ShapeDtypeStruct((E, D), f32),
    )(Xd, a, W_att, b_att.reshape(1, D), WihT, bih.reshape(1, 3 * D),
      WhhT, bhh.reshape(1, 3 * D))


def _tc_prep1_body(x1_ref, wa_ref, wb_ref, bal_ref, p1_ref, p2_ref):
    x1 = x1_ref[...]
    p1_ref[...] = _dot(x1, wa_ref[...]) + bal_ref[...]
    p2_ref[...] = _dot(x1, wb_ref[...])


def _tc_prep1(x1, Wa1, Wb1, b_align1):
    return pl.pallas_call(
        _tc_prep1_body,
        out_shape=(
            jax.ShapeDtypeStruct((N, 1), f32),
            jax.ShapeDtypeStruct((N, 1), f32),
        ),
    )(x1, Wa1, Wb1, b_align1.reshape(1, 1))


def kernel(x, edge_index, edge_attr,
           W_atom0, b_atom0, W_nb0, b_nb0, W_align0, b_align0, W_att0, b_att0,
           Wih0, Whh0, bih0, bhh0,
           W_align1, b_align1, W_att1, b_att1, Wih1, Whh1, bih1, bhh1):
    src = edge_index[0]
    dst = edge_index[1]
    zeros_n = jnp.zeros((N,), f32)
    zeros_nd = jnp.zeros((N, D), f32)

    u, v, p = _tc_prep0(x, W_atom0, b_atom0, W_nb0[:D], W_align0[:D])
    Vsrc, Pdst = _sc_gather0(v, p.reshape(N), src, dst)
    w = _tc_edge0a(Vsrc, edge_attr.T, Pdst.reshape(E // BE, BEH, 128), W_nb0[D:], b_nb0,
                   W_align0[D:], b_align0)
    Adst, Udst = _sc_seg_gather(w.reshape(E), src, dst, u, zeros_n)
    M = _tc_edge_msg(Udst, Adst.reshape(E // BE, BEH, 128), W_att0, b_att0,
                     Wih0.T, bih0, Whh0.T, bhh0)
    x1 = _sc_scatter(M, dst, zeros_nd)

    p1, p2 = _tc_prep1(x1, W_align1[:D], W_align1[D:], b_align1)
    A1, X1dst = _sc_edge1a(p1.reshape(N), p2.reshape(N), src, dst, x1, zeros_n)
    M1 = _tc_edge_msg(X1dst, A1.reshape(E // BE, BEH, 128), W_att1, b_att1,
                      Wih1.T, bih1, Whh1.T, bhh1)
    return _sc_scatter(M1, dst, zeros_nd)


# speedup vs baseline: 7.6872x; 1.0930x over previous
import functools

import jax
import jax.numpy as jnp
from jax import lax
from jax.experimental import pallas as pl
from jax.experimental.pallas import tpu as pltpu
from jax.experimental.pallas import tpu_sc as plsc

N = 10000
E = 320000
D = 128
NC = 2
NS = 16
NW = NC * NS
EW = E // NW
ET = E // NS
SUB = 80
CB = 400
NSUB = CB // SUB

_mesh = plsc.VectorSubcoreMesh(core_axis_name="c", subcore_axis_name="s")

f32 = jnp.float32
i32 = jnp.int32


def _wid():
    return lax.axis_index("s") * NC + lax.axis_index("c")


def _stage_idx(big, two_d):
    for j in range(NSUB):
        for k in range(SUB // 16):
            two_d[j, pl.ds(k * 16, 16)] = big[pl.ds(j * SUB + k * 16, 16)]


@functools.partial(
    pl.kernel,
    out_type=(
        jax.ShapeDtypeStruct((E, D), f32),
        jax.ShapeDtypeStruct((E,), f32),
    ),
    mesh=_mesh,
    compiler_params=pltpu.CompilerParams(needs_layout_passes=False),
    scratch_types=[
        pltpu.VMEM((N,), f32),
        pltpu.VMEM((CB,), i32),
        pltpu.VMEM((CB,), i32),
        pltpu.VMEM((NSUB, SUB), i32),
        pltpu.VMEM((CB, D), f32),
        pltpu.VMEM((CB,), f32),
        pltpu.SemaphoreType.DMA,
    ],
)
def _sc_gather0(v_hbm, p_hbm, src_hbm, dst_hbm, vout, pout,
                ptab, srcbig, dstbig, srcv, rows, pbuf, sem):
    wid = _wid()
    pltpu.sync_copy(p_hbm, ptab)

    def outer(i, carry):
        base = wid * EW + i * CB
        pltpu.sync_copy(src_hbm.at[pl.ds(base, CB)], srcbig)
        pltpu.sync_copy(dst_hbm.at[pl.ds(base, CB)], dstbig)
        _stage_idx(srcbig, srcv)
        descs = []
        for j in range(NSUB):
            descs.append(pltpu.async_copy(
                v_hbm.at[srcv.at[j]], rows.at[pl.ds(j * SUB, SUB)], sem))
        for k in range(CB // 16):
            idx = dstbig[pl.ds(k * 16, 16)]
            pbuf[pl.ds(k * 16, 16)] = plsc.load_gather(ptab, [idx])
        for d in descs:
            d.wait()
        pltpu.sync_copy(rows, vout.at[pl.ds(base, CB)])
        pltpu.sync_copy(pbuf, pout.at[pl.ds(base, CB)])
        return carry

    lax.fori_loop(0, EW // CB, outer, 0)


@functools.partial(
    pl.kernel,
    out_type=(
        jax.ShapeDtypeStruct((E,), f32),
        jax.ShapeDtypeStruct((E, D), f32),
    ),
    mesh=_mesh,
    compiler_params=pltpu.CompilerParams(needs_layout_passes=False),
    scratch_types=[
        pltpu.VMEM_SHARED((N,), f32),
        pltpu.VMEM((N,), f32),
        pltpu.VMEM((CB,), f32),
        pltpu.VMEM((CB,), i32),
        pltpu.VMEM((CB,), i32),
        pltpu.VMEM((NSUB, SUB), i32),
        pltpu.VMEM((NSUB, SUB), i32),
        pltpu.VMEM((CB, D), f32),
        pltpu.VMEM((CB,), f32),
        pltpu.SemaphoreType.DMA,
    ],
)
def _sc_seg_gather(w_hbm, src_hbm, dst_hbm, tab_hbm, zeros_hbm,
                   ssrc_out, rows_out,
                   S_sh, stile, wv, srcbig, dstbig, srcv, dstv, rows, sbuf,
                   sem):
    sid = lax.axis_index("s")
    wid = _wid()

    @pl.when(sid == 0)
    def _():
        pltpu.sync_copy(zeros_hbm, S_sh)

    plsc.subcore_barrier()

    def ph1(i, carry):
        base = sid * ET + i * CB
        pltpu.sync_copy(w_hbm.at[pl.ds(base, CB)], wv)
        pltpu.sync_copy(src_hbm.at[pl.ds(base, CB)], srcbig)
        _stage_idx(srcbig, srcv)
        descs = []
        for j in range(NSUB):
            descs.append(pltpu.async_copy(
                wv.at[pl.ds(j * SUB, SUB)], S_sh.at[srcv.at[j]], sem,
                add=True))
        for d in descs:
            d.wait()
        return carry

    lax.fori_loop(0, ET // CB, ph1, 0)
    plsc.subcore_barrier()
    pltpu.sync_copy(S_sh, stile)

    def ph2(i, carry):
        base = wid * EW + i * CB
        pltpu.sync_copy(src_hbm.at[pl.ds(base, CB)], srcbig)
        pltpu.sync_copy(dst_hbm.at[pl.ds(base, CB)], dstbig)
        pltpu.sync_copy(w_hbm.at[pl.ds(base, CB)], wv)
        _stage_idx(dstbig, dstv)
        descs = []
        for j in range(NSUB):
            descs.append(pltpu.async_copy(
                tab_hbm.at[dstv.at[j]], rows.at[pl.ds(j * SUB, SUB)], sem))
        for k in range(CB // 16):
            idx = srcbig[pl.ds(k * 16, 16)]
            sv = plsc.load_gather(stile, [idx])
            sbuf[pl.ds(k * 16, 16)] = wv[pl.ds(k * 16, 16)] / (sv + 1e-16)
        for d in descs:
            d.wait()
        pltpu.sync_copy(rows, rows_out.at[pl.ds(base, CB)])
        pltpu.sync_copy(sbuf, ssrc_out.at[pl.ds(base, CB)])
        return carry

    lax.fori_loop(0, EW // CB, ph2, 0)


NH = N // 2
NHP = NH + 8
CBS = 160
NSS = CBS // SUB
NITS = ET // CBS


@functools.partial(
    pl.kernel,
    out_type=jax.ShapeDtypeStruct((N, D), f32),
    mesh=_mesh,
    compiler_params=pltpu.CompilerParams(needs_layout_passes=False),
    scratch_types=[
        pltpu.VMEM_SHARED((NHP, D), f32),
        pltpu.VMEM((CBS, D), f32),
        pltpu.VMEM((CBS, D), f32),
        pltpu.VMEM((CBS,), i32),
        pltpu.VMEM((CBS,), i32),
        pltpu.VMEM((NSS, SUB), i32),
        pltpu.VMEM((NSS, SUB), i32),
        pltpu.SemaphoreType.DMA,
        pltpu.SemaphoreType.DMA,
        pltpu.SemaphoreType.DMA,
        pltpu.SemaphoreType.DMA,
    ],
)
def _sc_scatter(m_hbm, dst_hbm, zeros_hbm, out,
                X_sh, rowsA, rowsB, dstbigA, dstbigB, dstvA, dstvB,
                lsemA, lsemB, ssemA, ssemB):
    cid = lax.axis_index("c")
    sid = lax.axis_index("s")
    lo = cid * NH
    bufs = ((rowsA, dstbigA, dstvA, lsemA, ssemA),
            (rowsB, dstbigB, dstvB, lsemB, ssemB))

    @pl.when(sid == 0)
    def _():
        pltpu.sync_copy(zeros_hbm.at[pl.ds(0, NHP)], X_sh)

    plsc.subcore_barrier()

    base0 = sid * ET
    pltpu.async_copy(m_hbm.at[pl.ds(base0, CBS)], rowsA, lsemA)
    pltpu.async_copy(dst_hbm.at[pl.ds(base0, CBS)], dstbigA, lsemA)

    def chunk(c, b_):
        rows, dstbig, dstv, lsem, ssem = bufs[b_]
        rowsN, dstbigN, dstvN, lsemN, ssemN = bufs[1 - b_]

        @pl.when(c >= 1)
        def _():
            for j in range(NSS):
                pltpu.make_async_copy(
                    rowsN.at[pl.ds(j * SUB, SUB)],
                    X_sh.at[dstvN.at[j]], ssemN).wait()

        @pl.when(c + 1 < NITS)
        def _():
            nbase = sid * ET + (c + 1) * CBS
            pltpu.async_copy(m_hbm.at[pl.ds(nbase, CBS)], rowsN, lsemN)
            pltpu.async_copy(dst_hbm.at[pl.ds(nbase, CBS)], dstbigN, lsemN)

        pltpu.make_async_copy(m_hbm.at[pl.ds(0, CBS)], rows, lsem).wait()
        pltpu.make_async_copy(dst_hbm.at[pl.ds(0, CBS)], dstbig, lsem).wait()
        for k in range(CBS // 16):
            t = dstbig[pl.ds(k * 16, 16)] - lo
            t = jnp.where((t >= 0) & (t < NH), t, NH)
            dstv[k // (SUB // 16), pl.ds((k % (SUB // 16)) * 16, 16)] = t
        for j in range(NSS):
            pltpu.async_copy(rows.at[pl.ds(j * SUB, SUB)],
                             X_sh.at[dstv.at[j]], ssem, add=True)

    def body(c, carry):
        for b_ in (0, 1):
            @pl.when(lax.rem(c, 2) == b_)
            def _(b_=b_):
                chunk(c, b_)
        return carry

    lax.fori_loop(0, NITS, body, 0)
    rowsL, dstbigL, dstvL, lsemL, ssemL = bufs[(NITS - 1) % 2]
    for j in range(NSS):
        pltpu.make_async_copy(rowsL.at[pl.ds(j * SUB, SUB)],
                              X_sh.at[dstvL.at[j]], ssemL).wait()
    plsc.subcore_barrier()

    @pl.when(sid == 0)
    def _():
        pltpu.sync_copy(X_sh.at[pl.ds(0, NH)], out.at[pl.ds(lo, NH)])


@functools.partial(
    pl.kernel,
    out_type=(
        jax.ShapeDtypeStruct((E,), f32),
        jax.ShapeDtypeStruct((E, D), f32),
    ),
    mesh=_mesh,
    compiler_params=pltpu.CompilerParams(needs_layout_passes=False),
    scratch_types=[
        pltpu.VMEM_SHARED((N,), f32),
        pltpu.VMEM((N,), f32),
        pltpu.VMEM((N,), f32),
        pltpu.VMEM((N,), f32),
        pltpu.VMEM((CB,), f32),
        pltpu.VMEM((CB,), i32),
        pltpu.VMEM((CB,), i32),
        pltpu.VMEM((NSUB, SUB), i32),
        pltpu.VMEM((NSUB, SUB), i32),
        pltpu.VMEM((CB, D), f32),
        pltpu.VMEM((CB,), f32),
        pltpu.SemaphoreType.DMA,
    ],
)
def _sc_edge1a(p1_hbm, p2_hbm, src_hbm, dst_hbm, x1_hbm, zeros_hbm,
               ssrc_out, rows_out,
               S_sh, p1t, p2t, stile, wv, srcbig, dstbig, srcv, dstv, rows,
               sbuf, sem):
    cid = lax.axis_index("c")
    sid = lax.axis_index("s")
    wid = _wid()
    pltpu.sync_copy(p1_hbm, p1t)
    pltpu.sync_copy(p2_hbm, p2t)

    @pl.when(sid == 0)
    def _():
        pltpu.sync_copy(zeros_hbm, S_sh)

    plsc.subcore_barrier()

    def ph1(i, carry):
        base = sid * ET + i * CB
        pltpu.sync_copy(src_hbm.at[pl.ds(base, CB)], srcbig)
        pltpu.sync_copy(dst_hbm.at[pl.ds(base, CB)], dstbig)
        _stage_idx(srcbig, srcv)
        for k in range(CB // 16):
            idxd = dstbig[pl.ds(k * 16, 16)]
            idxs = srcbig[pl.ds(k * 16, 16)]
            t = plsc.load_gather(p1t, [idxd]) + plsc.load_gather(p2t, [idxs])
            t = jnp.where(t >= 0.0, t, 0.01 * t)
            wv[pl.ds(k * 16, 16)] = jnp.exp(t)
        descs = []
        for j in range(NSUB):
            descs.append(pltpu.async_copy(
                wv.at[pl.ds(j * SUB, SUB)], S_sh.at[srcv.at[j]], sem,
                add=True))
        for d in descs:
            d.wait()
        return carry

    lax.fori_loop(0, ET // CB, ph1, 0)
    plsc.subcore_barrier()
    pltpu.sync_copy(S_sh, stile)

    def ph2(i, carry):
        base = wid * EW + i * CB
        pltpu.sync_copy(src_hbm.at[pl.ds(base, CB)], srcbig)
        pltpu.sync_copy(dst_hbm.at[pl.ds(base, CB)], dstbig)
        _stage_idx(dstbig, dstv)
        descs = []
        for j in range(NSUB):
            descs.append(pltpu.async_copy(
                x1_hbm.at[dstv.at[j]], rows.at[pl.ds(j * SUB, SUB)], sem))
        for k in range(CB // 16):
            idxd = dstbig[pl.ds(k * 16, 16)]
            idxs = srcbig[pl.ds(k * 16, 16)]
            t = plsc.load_gather(p1t, [idxd]) + plsc.load_gather(p2t, [idxs])
            t = jnp.where(t >= 0.0, t, 0.01 * t)
            sv = plsc.load_gather(stile, [idxs])
            sbuf[pl.ds(k * 16, 16)] = jnp.exp(t) / (sv + 1e-16)
        for d in descs:
            d.wait()
        pltpu.sync_copy(rows, rows_out.at[pl.ds(base, CB)])
        pltpu.sync_copy(sbuf, ssrc_out.at[pl.ds(base, CB)])
        return carry

    lax.fori_loop(0, EW // CB, ph2, 0)


def _leaky(t):
    return jnp.where(t >= 0.0, t, 0.01 * t)


def _dot(a, b):
    return jnp.dot(a, b, preferred_element_type=f32)


def _tc_prep0_body(x_ref, wa_ref, ba_ref, wn_ref, wal_ref, u_ref, v_ref, p_ref):
    x = x_ref[...]
    u = _leaky(_dot(x, wa_ref[...]) + ba_ref[...])
    u_ref[...] = u
    v_ref[...] = _dot(x, wn_ref[...])
    p_ref[...] = _dot(u, wal_ref[...])


def _tc_prep0(x, W_atom0, b_atom0, Wnb1, Wal0a):
    return pl.pallas_call(
        _tc_prep0_body,
        out_shape=(
            jax.ShapeDtypeStruct((N, D), f32),
            jax.ShapeDtypeStruct((N, D), f32),
            jax.ShapeDtypeStruct((N, 1), f32),
        ),
    )(x, W_atom0, b_atom0.reshape(1, D), Wnb1, Wal0a)


BE = 2560
BEH = BE // 128


def _tc_edge0a_body(vs_ref, ea_ref, pd_ref, wn2_ref, bn_ref, wb_ref, bal_ref,
                    w_ref):
    ea = lax.dot_general(ea_ref[...], wn2_ref[...], (((0,), (0,)), ((), ())),
                         preferred_element_type=f32)
    xj = _leaky(vs_ref[...] + ea + bn_ref[...])
    ej = _dot(xj, wb_ref[...])
    pdt = jnp.transpose(pd_ref[0])
    pdcol = jnp.concatenate([pdt[:, r:r + 1] for r in range(BEH)], axis=0)
    e = _leaky(pdcol + ej + bal_ref[...])
    w = jnp.exp(e)
    rows = [jnp.transpose(w[r * 128:(r + 1) * 128, :]) for r in range(BEH)]
    w_ref[...] = jnp.reshape(jnp.concatenate(rows, axis=0), (1, BEH, 128))


def _tc_edge0a(Vsrc, eaT, Pdst, Wnb2, b_nb0, Wb0, b_align0):
    full = lambda s: pl.BlockSpec(s, lambda i: (0, 0))
    return pl.pallas_call(
        _tc_edge0a_body,
        grid=(E // BE,),
        in_specs=[
            pl.BlockSpec((BE, D), lambda i: (i, 0)),
            pl.BlockSpec((16, BE), lambda i: (0, i)),
            pl.BlockSpec((1, BEH, 128), lambda i: (i, 0, 0)),
            full((16, D)),
            full((1, D)),
            full((D, 1)),
            full((1, 1)),
        ],
        out_specs=pl.BlockSpec((1, BEH, 128), lambda i: (i, 0, 0)),
        out_shape=jax.ShapeDtypeStruct((E // BE, BEH, 128), f32),
    )(Vsrc, eaT, Pdst, Wnb2, b_nb0.reshape(1, D), Wb0,
      b_align0.reshape(1, 1))


def _tc_edge_msg_body(xi_ref, a_ref, watt_ref, batt_ref,
                      wih_ref, bih_ref, whh_ref, bhh_ref, m_ref):
    x_i = xi_ref[...]
    at = jnp.transpose(a_ref[0])
    q = _dot(x_i, watt_ref[...]) + batt_ref[...]
    aq = jnp.concatenate(
        [q[r * 128:(r + 1) * 128, :] * at[:, r:r + 1] for r in range(BEH)],
        axis=0)
    c = jnp.where(aq > 0.0, aq, jnp.exp(aq) - 1.0)
    gi = _dot(c, wih_ref[...]) + bih_ref[...]
    gh = _dot(x_i, whh_ref[...]) + bhh_ref[...]
    r = jax.nn.sigmoid(gi[:, :D] + gh[:, :D])
    z = jax.nn.sigmoid(gi[:, D:2 * D] + gh[:, D:2 * D])
    n = jnp.tanh(gi[:, 2 * D:] + r * gh[:, 2 * D:])
    m_ref[...] = (1.0 - z) * n + z * x_i


def _tc_edge_msg(Xd, a, W_att, b_att, WihT, bih, WhhT, bhh):
    full = lambda s: pl.BlockSpec(s, lambda i: (0, 0))
    return pl.pallas_call(
        _tc_edge_msg_body,
        grid=(E // BE,),
        in_specs=[
            pl.BlockSpec((BE, D), lambda i: (i, 0)),
            pl.BlockSpec((1, BEH, 128), lambda i: (i, 0, 0)),
            full((D, D)),
            full((1, D)),
            full((D, 3 * D)),
            full((1, 3 * D)),
            full((D, 3 * D)),
            full((1, 3 * D)),
        ],
        out_specs=pl.BlockSpec((BE, D), lambda i: (i, 0)),
        out_shape=jax.ShapeDtypeStruct((E, D), f32),
    )(Xd, a, W_att, b_att.reshape(1, D), WihT, bih.reshape(1, 3 * D),
      WhhT, bhh.reshape(1, 3 * D))


def _tc_prep1_body(x1_ref, wa_ref, wb_ref, bal_ref, p1_ref, p2_ref):
    x1 = x1_ref[...]
    p1_ref[...] = _dot(x1, wa_ref[...]) + bal_ref[...]
    p2_ref[...] = _dot(x1, wb_ref[...])


def _tc_prep1(x1, Wa1, Wb1, b_align1):
    return pl.pallas_call(
        _tc_prep1_body,
        out_shape=(
            jax.ShapeDtypeStruct((N, 1), f32),
            jax.ShapeDtypeStruct((N, 1), f32),
        ),
    )(x1, Wa1, Wb1, b_align1.reshape(1, 1))


def kernel(x, edge_index, edge_attr,
           W_atom0, b_atom0, W_nb0, b_nb0, W_align0, b_align0, W_att0, b_att0,
           Wih0, Whh0, bih0, bhh0,
           W_align1, b_align1, W_att1, b_att1, Wih1, Whh1, bih1, bhh1):
    src = edge_index[0]
    dst = edge_index[1]
    zeros_n = jnp.zeros((N,), f32)
    zeros_nd = jnp.zeros((N, D), f32)

    u, v, p = _tc_prep0(x, W_atom0, b_atom0, W_nb0[:D], W_align0[:D])
    Vsrc, Pdst = _sc_gather0(v, p.reshape(N), src, dst)
    w = _tc_edge0a(Vsrc, edge_attr.T, Pdst.reshape(E // BE, BEH, 128), W_nb0[D:], b_nb0,
                   W_align0[D:], b_align0)
    Adst, Udst = _sc_seg_gather(w.reshape(E), src, dst, u, zeros_n)
    M = _tc_edge_msg(Udst, Adst.reshape(E // BE, BEH, 128), W_att0, b_att0,
                     Wih0.T, bih0, Whh0.T, bhh0)
    x1 = _sc_scatter(M, dst, zeros_nd)

    p1, p2 = _tc_prep1(x1, W_align1[:D], W_align1[D:], b_align1)
    A1, X1dst = _sc_edge1a(p1.reshape(N), p2.reshape(N), src, dst, x1, zeros_n)
    M1 = _tc_edge_msg(X1dst, A1.reshape(E // BE, BEH, 128), W_att1, b_att1,
                      Wih1.T, bih1, Whh1.T, bhh1)
    return _sc_scatter(M1, dst, zeros_nd)


# speedup vs baseline: 7.8091x; 1.0159x over previous
import functools

import jax
import jax.numpy as jnp
from jax import lax
from jax.experimental import pallas as pl
from jax.experimental.pallas import tpu as pltpu
from jax.experimental.pallas import tpu_sc as plsc

N = 10000
E = 320000
D = 128
NC = 2
NS = 16
NW = NC * NS
EW = E // NW
ET = E // NS
SUB = 80
CB = 400
NSUB = CB // SUB

_mesh = plsc.VectorSubcoreMesh(core_axis_name="c", subcore_axis_name="s")

f32 = jnp.float32
i32 = jnp.int32


def _wid():
    return lax.axis_index("s") * NC + lax.axis_index("c")


def _stage_idx(big, two_d):
    for j in range(NSUB):
        for k in range(SUB // 16):
            two_d[j, pl.ds(k * 16, 16)] = big[pl.ds(j * SUB + k * 16, 16)]


@functools.partial(
    pl.kernel,
    out_type=(
        jax.ShapeDtypeStruct((E, D), f32),
        jax.ShapeDtypeStruct((E,), f32),
    ),
    mesh=_mesh,
    compiler_params=pltpu.CompilerParams(needs_layout_passes=False),
    scratch_types=[
        pltpu.VMEM((N,), f32),
        pltpu.VMEM((CB,), i32),
        pltpu.VMEM((CB,), i32),
        pltpu.VMEM((NSUB, SUB), i32),
        pltpu.VMEM((CB, D), f32),
        pltpu.VMEM((CB,), f32),
        pltpu.SemaphoreType.DMA,
    ],
)
def _sc_gather0(v_hbm, p_hbm, src_hbm, dst_hbm, vout, pout,
                ptab, srcbig, dstbig, srcv, rows, pbuf, sem):
    wid = _wid()
    pltpu.sync_copy(p_hbm, ptab)

    def outer(i, carry):
        base = wid * EW + i * CB
        pltpu.sync_copy(src_hbm.at[pl.ds(base, CB)], srcbig)
        pltpu.sync_copy(dst_hbm.at[pl.ds(base, CB)], dstbig)
        _stage_idx(srcbig, srcv)
        descs = []
        for j in range(NSUB):
            descs.append(pltpu.async_copy(
                v_hbm.at[srcv.at[j]], rows.at[pl.ds(j * SUB, SUB)], sem))
        for k in range(CB // 16):
            idx = dstbig[pl.ds(k * 16, 16)]
            pbuf[pl.ds(k * 16, 16)] = plsc.load_gather(ptab, [idx])
        for d in descs:
            d.wait()
        pltpu.sync_copy(rows, vout.at[pl.ds(base, CB)])
        pltpu.sync_copy(pbuf, pout.at[pl.ds(base, CB)])
        return carry

    lax.fori_loop(0, EW // CB, outer, 0)


@functools.partial(
    pl.kernel,
    out_type=(
        jax.ShapeDtypeStruct((E,), f32),
        jax.ShapeDtypeStruct((E, D), f32),
    ),
    mesh=_mesh,
    compiler_params=pltpu.CompilerParams(needs_layout_passes=False),
    scratch_types=[
        pltpu.VMEM_SHARED((N,), f32),
        pltpu.VMEM((N,), f32),
        pltpu.VMEM((CB,), f32),
        pltpu.VMEM((CB,), i32),
        pltpu.VMEM((CB,), i32),
        pltpu.VMEM((NSUB, SUB), i32),
        pltpu.VMEM((NSUB, SUB), i32),
        pltpu.VMEM((CB, D), f32),
        pltpu.VMEM((CB,), f32),
        pltpu.SemaphoreType.DMA,
    ],
)
def _sc_seg_gather(w_hbm, src_hbm, dst_hbm, tab_hbm, zeros_hbm,
                   ssrc_out, rows_out,
                   S_sh, stile, wv, srcbig, dstbig, srcv, dstv, rows, sbuf,
                   sem):
    sid = lax.axis_index("s")
    wid = _wid()

    @pl.when(sid == 0)
    def _():
        pltpu.sync_copy(zeros_hbm, S_sh)

    plsc.subcore_barrier()

    def ph1(i, carry):
        base = sid * ET + i * CB
        pltpu.sync_copy(w_hbm.at[pl.ds(base, CB)], wv)
        pltpu.sync_copy(src_hbm.at[pl.ds(base, CB)], srcbig)
        _stage_idx(srcbig, srcv)
        descs = []
        for j in range(NSUB):
            descs.append(pltpu.async_copy(
                wv.at[pl.ds(j * SUB, SUB)], S_sh.at[srcv.at[j]], sem,
                add=True))
        for d in descs:
            d.wait()
        return carry

    lax.fori_loop(0, ET // CB, ph1, 0)
    plsc.subcore_barrier()
    pltpu.sync_copy(S_sh, stile)

    def ph2(i, carry):
        base = wid * EW + i * CB
        pltpu.sync_copy(src_hbm.at[pl.ds(base, CB)], srcbig)
        pltpu.sync_copy(dst_hbm.at[pl.ds(base, CB)], dstbig)
        pltpu.sync_copy(w_hbm.at[pl.ds(base, CB)], wv)
        _stage_idx(dstbig, dstv)
        descs = []
        for j in range(NSUB):
            descs.append(pltpu.async_copy(
                tab_hbm.at[dstv.at[j]], rows.at[pl.ds(j * SUB, SUB)], sem))
        for k in range(CB // 16):
            idx = srcbig[pl.ds(k * 16, 16)]
            sv = plsc.load_gather(stile, [idx])
            sbuf[pl.ds(k * 16, 16)] = wv[pl.ds(k * 16, 16)] / (sv + 1e-16)
        for d in descs:
            d.wait()
        pltpu.sync_copy(rows, rows_out.at[pl.ds(base, CB)])
        pltpu.sync_copy(sbuf, ssrc_out.at[pl.ds(base, CB)])
        return carry

    lax.fori_loop(0, EW // CB, ph2, 0)


NH = N // 2
NHP = NH + 8
CBS = 160
NSS = CBS // SUB
NITS = ET // CBS


@functools.partial(
    pl.kernel,
    out_type=jax.ShapeDtypeStruct((N, D), f32),
    mesh=_mesh,
    compiler_params=pltpu.CompilerParams(needs_layout_passes=False),
    scratch_types=[
        pltpu.VMEM_SHARED((NHP, D), f32),
        pltpu.VMEM((CBS, D), f32),
        pltpu.VMEM((CBS, D), f32),
        pltpu.VMEM((CBS,), i32),
        pltpu.VMEM((CBS,), i32),
        pltpu.VMEM((NSS, SUB), i32),
        pltpu.VMEM((NSS, SUB), i32),
        pltpu.SemaphoreType.DMA,
        pltpu.SemaphoreType.DMA,
        pltpu.SemaphoreType.DMA,
        pltpu.SemaphoreType.DMA,
    ],
)
def _sc_scatter(m_hbm, dst_hbm, zeros_hbm, out,
                X_sh, rowsA, rowsB, dstbigA, dstbigB, dstvA, dstvB,
                lsemA, lsemB, ssemA, ssemB):
    cid = lax.axis_index("c")
    sid = lax.axis_index("s")
    lo = cid * NH
    bufs = ((rowsA, dstbigA, dstvA, lsemA, ssemA),
            (rowsB, dstbigB, dstvB, lsemB, ssemB))

    @pl.when(sid == 0)
    def _():
        pltpu.sync_copy(zeros_hbm.at[pl.ds(0, NHP)], X_sh)

    plsc.subcore_barrier()

    base0 = sid * ET
    pltpu.async_copy(m_hbm.at[pl.ds(base0, CBS)], rowsA, lsemA)
    pltpu.async_copy(dst_hbm.at[pl.ds(base0, CBS)], dstbigA, lsemA)

    def chunk(c, b_):
        rows, dstbig, dstv, lsem, ssem = bufs[b_]
        rowsN, dstbigN, dstvN, lsemN, ssemN = bufs[1 - b_]

        @pl.when(c >= 1)
        def _():
            for j in range(NSS):
                pltpu.make_async_copy(
                    rowsN.at[pl.ds(j * SUB, SUB)],
                    X_sh.at[dstvN.at[j]], ssemN).wait()

        @pl.when(c + 1 < NITS)
        def _():
            nbase = sid * ET + (c + 1) * CBS
            pltpu.async_copy(m_hbm.at[pl.ds(nbase, CBS)], rowsN, lsemN)
            pltpu.async_copy(dst_hbm.at[pl.ds(nbase, CBS)], dstbigN, lsemN)

        pltpu.make_async_copy(m_hbm.at[pl.ds(0, CBS)], rows, lsem).wait()
        pltpu.make_async_copy(dst_hbm.at[pl.ds(0, CBS)], dstbig, lsem).wait()
        for k in range(CBS // 16):
            t = dstbig[pl.ds(k * 16, 16)] - lo
            t = jnp.where((t >= 0) & (t < NH), t, NH)
            dstv[k // (SUB // 16), pl.ds((k % (SUB // 16)) * 16, 16)] = t
        for j in range(NSS):
            pltpu.async_copy(rows.at[pl.ds(j * SUB, SUB)],
                             X_sh.at[dstv.at[j]], ssem, add=True)

    def body(c, carry):
        for b_ in (0, 1):
            @pl.when(lax.rem(c, 2) == b_)
            def _(b_=b_):
                chunk(c, b_)
        return carry

    lax.fori_loop(0, NITS, body, 0)
    rowsL, dstbigL, dstvL, lsemL, ssemL = bufs[(NITS - 1) % 2]
    for j in range(NSS):
        pltpu.make_async_copy(rowsL.at[pl.ds(j * SUB, SUB)],
                              X_sh.at[dstvL.at[j]], ssemL).wait()
    plsc.subcore_barrier()

    @pl.when(sid == 0)
    def _():
        pltpu.sync_copy(X_sh.at[pl.ds(0, NH)], out.at[pl.ds(lo, NH)])


@functools.partial(
    pl.kernel,
    out_type=(
        jax.ShapeDtypeStruct((E,), f32),
        jax.ShapeDtypeStruct((E, D), f32),
    ),
    mesh=_mesh,
    compiler_params=pltpu.CompilerParams(needs_layout_passes=False),
    scratch_types=[
        pltpu.VMEM_SHARED((N,), f32),
        pltpu.VMEM((N,), f32),
        pltpu.VMEM((N,), f32),
        pltpu.VMEM((N,), f32),
        pltpu.VMEM((CB,), f32),
        pltpu.VMEM((CB,), i32),
        pltpu.VMEM((CB,), i32),
        pltpu.VMEM((NSUB, SUB), i32),
        pltpu.VMEM((NSUB, SUB), i32),
        pltpu.VMEM((CB, D), f32),
        pltpu.VMEM((CB,), f32),
        pltpu.SemaphoreType.DMA,
    ],
)
def _sc_edge1a(p1_hbm, p2_hbm, src_hbm, dst_hbm, x1_hbm, zeros_hbm,
               ssrc_out, rows_out,
               S_sh, p1t, p2t, stile, wv, srcbig, dstbig, srcv, dstv, rows,
               sbuf, sem):
    cid = lax.axis_index("c")
    sid = lax.axis_index("s")
    wid = _wid()
    pltpu.sync_copy(p1_hbm, p1t)
    pltpu.sync_copy(p2_hbm, p2t)

    @pl.when(sid == 0)
    def _():
        pltpu.sync_copy(zeros_hbm, S_sh)

    plsc.subcore_barrier()

    def ph1(i, carry):
        base = sid * ET + i * CB
        pltpu.sync_copy(src_hbm.at[pl.ds(base, CB)], srcbig)
        pltpu.sync_copy(dst_hbm.at[pl.ds(base, CB)], dstbig)
        _stage_idx(srcbig, srcv)
        for k in range(CB // 16):
            idxd = dstbig[pl.ds(k * 16, 16)]
            idxs = srcbig[pl.ds(k * 16, 16)]
            t = plsc.load_gather(p1t, [idxd]) + plsc.load_gather(p2t, [idxs])
            t = jnp.where(t >= 0.0, t, 0.01 * t)
            wv[pl.ds(k * 16, 16)] = jnp.exp(t)
        descs = []
        for j in range(NSUB):
            descs.append(pltpu.async_copy(
                wv.at[pl.ds(j * SUB, SUB)], S_sh.at[srcv.at[j]], sem,
                add=True))
        for d in descs:
            d.wait()
        return carry

    lax.fori_loop(0, ET // CB, ph1, 0)
    plsc.subcore_barrier()
    pltpu.sync_copy(S_sh, stile)

    def ph2(i, carry):
        base = wid * EW + i * CB
        pltpu.sync_copy(src_hbm.at[pl.ds(base, CB)], srcbig)
        pltpu.sync_copy(dst_hbm.at[pl.ds(base, CB)], dstbig)
        _stage_idx(dstbig, dstv)
        descs = []
        for j in range(NSUB):
            descs.append(pltpu.async_copy(
                x1_hbm.at[dstv.at[j]], rows.at[pl.ds(j * SUB, SUB)], sem))
        for k in range(CB // 16):
            idxd = dstbig[pl.ds(k * 16, 16)]
            idxs = srcbig[pl.ds(k * 16, 16)]
            t = plsc.load_gather(p1t, [idxd]) + plsc.load_gather(p2t, [idxs])
            t = jnp.where(t >= 0.0, t, 0.01 * t)
            sv = plsc.load_gather(stile, [idxs])
            sbuf[pl.ds(k * 16, 16)] = jnp.exp(t) / (sv + 1e-16)
        for d in descs:
            d.wait()
        pltpu.sync_copy(rows, rows_out.at[pl.ds(base, CB)])
        pltpu.sync_copy(sbuf, ssrc_out.at[pl.ds(base, CB)])
        return carry

    lax.fori_loop(0, EW // CB, ph2, 0)


def _leaky(t):
    return jnp.where(t >= 0.0, t, 0.01 * t)


def _dot(a, b):
    return jnp.dot(a, b, preferred_element_type=f32)


def _tc_prep0_body(x_ref, wa_ref, ba_ref, wn_ref, wal_ref, u_ref, v_ref, p_ref):
    x = x_ref[...]
    u = _leaky(_dot(x, wa_ref[...]) + ba_ref[...])
    u_ref[...] = u
    v_ref[...] = _dot(x, wn_ref[...])
    p_ref[...] = _dot(u, wal_ref[...])


def _tc_prep0(x, W_atom0, b_atom0, Wnb1, Wal0a):
    return pl.pallas_call(
        _tc_prep0_body,
        out_shape=(
            jax.ShapeDtypeStruct((N, D), f32),
            jax.ShapeDtypeStruct((N, D), f32),
            jax.ShapeDtypeStruct((N, 1), f32),
        ),
    )(x, W_atom0, b_atom0.reshape(1, D), Wnb1, Wal0a)


BE = 3200
BEH = BE // 128


def _tc_edge0a_body(vs_ref, ea_ref, pd_ref, wn2_ref, bn_ref, wb_ref, bal_ref,
                    w_ref):
    ea = lax.dot_general(ea_ref[...], wn2_ref[...], (((0,), (0,)), ((), ())),
                         preferred_element_type=f32)
    xj = _leaky(vs_ref[...] + ea + bn_ref[...])
    ej = _dot(xj, wb_ref[...])
    pdt = jnp.transpose(pd_ref[0])
    pdcol = jnp.concatenate([pdt[:, r:r + 1] for r in range(BEH)], axis=0)
    e = _leaky(pdcol + ej + bal_ref[...])
    w = jnp.exp(e)
    rows = [jnp.transpose(w[r * 128:(r + 1) * 128, :]) for r in range(BEH)]
    w_ref[...] = jnp.reshape(jnp.concatenate(rows, axis=0), (1, BEH, 128))


def _tc_edge0a(Vsrc, eaT, Pdst, Wnb2, b_nb0, Wb0, b_align0):
    full = lambda s: pl.BlockSpec(s, lambda i: (0, 0))
    return pl.pallas_call(
        _tc_edge0a_body,
        grid=(E // BE,),
        compiler_params=pltpu.CompilerParams(
            fuse_transposed_lhs_in_matmul=True),
        in_specs=[
            pl.BlockSpec((BE, D), lambda i: (i, 0)),
            pl.BlockSpec((16, BE), lambda i: (0, i)),
            pl.BlockSpec((1, BEH, 128), lambda i: (i, 0, 0)),
            full((16, D)),
            full((1, D)),
            full((D, 1)),
            full((1, 1)),
        ],
        out_specs=pl.BlockSpec((1, BEH, 128), lambda i: (i, 0, 0)),
        out_shape=jax.ShapeDtypeStruct((E // BE, BEH, 128), f32),
    )(Vsrc, eaT, Pdst, Wnb2, b_nb0.reshape(1, D), Wb0,
      b_align0.reshape(1, 1))


def _tc_edge_msg_body(xi_ref, a_ref, watt_ref, batt_ref,
                      wih_ref, bih_ref, whh_ref, bhh_ref, m_ref):
    x_i = xi_ref[...]
    at = jnp.transpose(a_ref[0])
    q = _dot(x_i, watt_ref[...]) + batt_ref[...]
    aq = jnp.concatenate(
        [q[r * 128:(r + 1) * 128, :] * at[:, r:r + 1] for r in range(BEH)],
        axis=0)
    c = jnp.where(aq > 0.0, aq, jnp.exp(aq) - 1.0)
    gi = _dot(c, wih_ref[...]) + bih_ref[...]
    gh = _dot(x_i, whh_ref[...]) + bhh_ref[...]
    r = jax.nn.sigmoid(gi[:, :D] + gh[:, :D])
    z = jax.nn.sigmoid(gi[:, D:2 * D] + gh[:, D:2 * D])
    n = jnp.tanh(gi[:, 2 * D:] + r * gh[:, 2 * D:])
    m_ref[...] = (1.0 - z) * n + z * x_i


def _tc_edge_msg(Xd, a, W_att, b_att, WihT, bih, WhhT, bhh):
    full = lambda s: pl.BlockSpec(s, lambda i: (0, 0))
    return pl.pallas_call(
        _tc_edge_msg_body,
        grid=(E // BE,),
        in_specs=[
            pl.BlockSpec((BE, D), lambda i: (i, 0)),
            pl.BlockSpec((1, BEH, 128), lambda i: (i, 0, 0)),
            full((D, D)),
            full((1, D)),
            full((D, 3 * D)),
            full((1, 3 * D)),
            full((D, 3 * D)),
            full((1, 3 * D)),
        ],
        out_specs=pl.BlockSpec((BE, D), lambda i: (i, 0)),
        out_shape=jax.ShapeDtypeStruct((E, D), f32),
    )(Xd, a, W_att, b_att.reshape(1, D), WihT, bih.reshape(1, 3 * D),
      WhhT, bhh.reshape(1, 3 * D))


def _tc_prep1_body(x1_ref, wa_ref, wb_ref, bal_ref, p1_ref, p2_ref):
    x1 = x1_ref[...]
    p1_ref[...] = _dot(x1, wa_ref[...]) + bal_ref[...]
    p2_ref[...] = _dot(x1, wb_ref[...])


def _tc_prep1(x1, Wa1, Wb1, b_align1):
    return pl.pallas_call(
        _tc_prep1_body,
        out_shape=(
            jax.ShapeDtypeStruct((N, 1), f32),
            jax.ShapeDtypeStruct((N, 1), f32),
        ),
    )(x1, Wa1, Wb1, b_align1.reshape(1, 1))


def kernel(x, edge_index, edge_attr,
           W_atom0, b_atom0, W_nb0, b_nb0, W_align0, b_align0, W_att0, b_att0,
           Wih0, Whh0, bih0, bhh0,
           W_align1, b_align1, W_att1, b_att1, Wih1, Whh1, bih1, bhh1):
    src = edge_index[0]
    dst = edge_index[1]
    zeros_n = jnp.zeros((N,), f32)
    zeros_nd = jnp.zeros((N, D), f32)

    u, v, p = _tc_prep0(x, W_atom0, b_atom0, W_nb0[:D], W_align0[:D])
    Vsrc, Pdst = _sc_gather0(v, p.reshape(N), src, dst)
    w = _tc_edge0a(Vsrc, edge_attr.T, Pdst.reshape(E // BE, BEH, 128), W_nb0[D:], b_nb0,
                   W_align0[D:], b_align0)
    Adst, Udst = _sc_seg_gather(w.reshape(E), src, dst, u, zeros_n)
    M = _tc_edge_msg(Udst, Adst.reshape(E // BE, BEH, 128), W_att0, b_att0,
                     Wih0.T, bih0, Whh0.T, bhh0)
    x1 = _sc_scatter(M, dst, zeros_nd)

    p1, p2 = _tc_prep1(x1, W_align1[:D], W_align1[D:], b_align1)
    A1, X1dst = _sc_edge1a(p1.reshape(N), p2.reshape(N), src, dst, x1, zeros_n)
    M1 = _tc_edge_msg(X1dst, A1.reshape(E // BE, BEH, 128), W_att1, b_att1,
                      Wih1.T, bih1, Whh1.T, bhh1)
    return _sc_scatter(M1, dst, zeros_nd)


# speedup vs baseline: 8.0504x; 1.0309x over previous
import functools

import jax
import jax.numpy as jnp
from jax import lax
from jax.experimental import pallas as pl
from jax.experimental.pallas import tpu as pltpu
from jax.experimental.pallas import tpu_sc as plsc

N = 10000
E = 320000
D = 128
NC = 2
NS = 16
NW = NC * NS
EW = E // NW
ET = E // NS
SUB = 80
CB = 400
NSUB = CB // SUB

_mesh = plsc.VectorSubcoreMesh(core_axis_name="c", subcore_axis_name="s")

f32 = jnp.float32
i32 = jnp.int32


def _wid():
    return lax.axis_index("s") * NC + lax.axis_index("c")


def _stage_idx(big, two_d):
    for j in range(NSUB):
        for k in range(SUB // 16):
            two_d[j, pl.ds(k * 16, 16)] = big[pl.ds(j * SUB + k * 16, 16)]


H0 = 153600
H1 = E - H0


def _make_gather0(eoff, esz):
    ewh = esz // NW

    @functools.partial(
        pl.kernel,
        out_type=(
            jax.ShapeDtypeStruct((esz, D), f32),
            jax.ShapeDtypeStruct((esz,), f32),
        ),
        mesh=_mesh,
        compiler_params=pltpu.CompilerParams(needs_layout_passes=False),
        scratch_types=[
            pltpu.VMEM((N,), f32),
            pltpu.VMEM((CB,), i32),
            pltpu.VMEM((CB,), i32),
            pltpu.VMEM((NSUB, SUB), i32),
            pltpu.VMEM((CB, D), f32),
            pltpu.VMEM((CB,), f32),
            pltpu.SemaphoreType.DMA,
        ],
    )
    def k(v_hbm, p_hbm, src_hbm, dst_hbm, vout, pout,
          ptab, srcbig, dstbig, srcv, rows, pbuf, sem):
        wid = _wid()
        pltpu.sync_copy(p_hbm, ptab)

        def outer(i, carry):
            lbase = wid * ewh + i * CB
            gbase = eoff + lbase
            pltpu.sync_copy(src_hbm.at[pl.ds(gbase, CB)], srcbig)
            pltpu.sync_copy(dst_hbm.at[pl.ds(gbase, CB)], dstbig)
            _stage_idx(srcbig, srcv)
            descs = []
            for j in range(NSUB):
                descs.append(pltpu.async_copy(
                    v_hbm.at[srcv.at[j]], rows.at[pl.ds(j * SUB, SUB)], sem))
            for kk in range(CB // 16):
                idx = dstbig[pl.ds(kk * 16, 16)]
                pbuf[pl.ds(kk * 16, 16)] = plsc.load_gather(ptab, [idx])
            for d in descs:
                d.wait()
            pltpu.sync_copy(rows, vout.at[pl.ds(lbase, CB)])
            pltpu.sync_copy(pbuf, pout.at[pl.ds(lbase, CB)])
            return carry

        lax.fori_loop(0, ewh // CB, outer, 0)

    return k


_sc_gather0_h = (_make_gather0(0, H0), _make_gather0(H0, H1))


@functools.partial(
    pl.kernel,
    out_type=(
        jax.ShapeDtypeStruct((E,), f32),
        jax.ShapeDtypeStruct((E, D), f32),
    ),
    mesh=_mesh,
    compiler_params=pltpu.CompilerParams(needs_layout_passes=False),
    scratch_types=[
        pltpu.VMEM_SHARED((N,), f32),
        pltpu.VMEM((N,), f32),
        pltpu.VMEM((CB,), f32),
        pltpu.VMEM((CB,), i32),
        pltpu.VMEM((CB,), i32),
        pltpu.VMEM((NSUB, SUB), i32),
        pltpu.VMEM((NSUB, SUB), i32),
        pltpu.VMEM((CB, D), f32),
        pltpu.VMEM((CB,), f32),
        pltpu.SemaphoreType.DMA,
    ],
)
def _sc_seg_gather(w0_hbm, w1_hbm, src_hbm, dst_hbm, tab_hbm, zeros_hbm,
                   ssrc_out, rows_out,
                   S_sh, stile, wv, srcbig, dstbig, srcv, dstv, rows, sbuf,
                   sem):
    sid = lax.axis_index("s")
    wid = _wid()

    @pl.when(sid == 0)
    def _():
        pltpu.sync_copy(zeros_hbm, S_sh)

    plsc.subcore_barrier()

    def ph1a(i, carry):
        lbase = sid * (H0 // NS) + i * CB
        pltpu.sync_copy(w0_hbm.at[pl.ds(lbase, CB)], wv)
        pltpu.sync_copy(src_hbm.at[pl.ds(lbase, CB)], srcbig)
        _stage_idx(srcbig, srcv)
        descs = []
        for j in range(NSUB):
            descs.append(pltpu.async_copy(
                wv.at[pl.ds(j * SUB, SUB)], S_sh.at[srcv.at[j]], sem,
                add=True))
        for d in descs:
            d.wait()
        return carry

    def ph1b(i, carry):
        lbase = sid * (H1 // NS) + i * CB
        pltpu.sync_copy(w1_hbm.at[pl.ds(lbase, CB)], wv)
        pltpu.sync_copy(src_hbm.at[pl.ds(H0 + lbase, CB)], srcbig)
        _stage_idx(srcbig, srcv)
        descs = []
        for j in range(NSUB):
            descs.append(pltpu.async_copy(
                wv.at[pl.ds(j * SUB, SUB)], S_sh.at[srcv.at[j]], sem,
                add=True))
        for d in descs:
            d.wait()
        return carry

    lax.fori_loop(0, H0 // NS // CB, ph1a, 0)
    lax.fori_loop(0, H1 // NS // CB, ph1b, 0)
    plsc.subcore_barrier()
    pltpu.sync_copy(S_sh, stile)

    def ph2(i, carry):
        base = wid * EW + i * CB
        pltpu.sync_copy(src_hbm.at[pl.ds(base, CB)], srcbig)
        pltpu.sync_copy(dst_hbm.at[pl.ds(base, CB)], dstbig)

        @pl.when(base < H0)
        def _():
            pltpu.sync_copy(w0_hbm.at[pl.ds(base, CB)], wv)

        @pl.when(base >= H0)
        def _():
            pltpu.sync_copy(w1_hbm.at[pl.ds(base - H0, CB)], wv)

        _stage_idx(dstbig, dstv)
        descs = []
        for j in range(NSUB):
            descs.append(pltpu.async_copy(
                tab_hbm.at[dstv.at[j]], rows.at[pl.ds(j * SUB, SUB)], sem))
        for k in range(CB // 16):
            idx = srcbig[pl.ds(k * 16, 16)]
            sv = plsc.load_gather(stile, [idx])
            sbuf[pl.ds(k * 16, 16)] = wv[pl.ds(k * 16, 16)] / (sv + 1e-16)
        for d in descs:
            d.wait()
        pltpu.sync_copy(rows, rows_out.at[pl.ds(base, CB)])
        pltpu.sync_copy(sbuf, ssrc_out.at[pl.ds(base, CB)])
        return carry

    lax.fori_loop(0, EW // CB, ph2, 0)


NH = N // 2
NHP = NH + 8
CBS = 160
NSS = CBS // SUB
NITS = ET // CBS


@functools.partial(
    pl.kernel,
    out_type=jax.ShapeDtypeStruct((N, D), f32),
    mesh=_mesh,
    compiler_params=pltpu.CompilerParams(needs_layout_passes=False),
    scratch_types=[
        pltpu.VMEM_SHARED((NHP, D), f32),
        pltpu.VMEM((CBS, D), f32),
        pltpu.VMEM((CBS, D), f32),
        pltpu.VMEM((CBS,), i32),
        pltpu.VMEM((CBS,), i32),
        pltpu.VMEM((NSS, SUB), i32),
        pltpu.VMEM((NSS, SUB), i32),
        pltpu.SemaphoreType.DMA,
        pltpu.SemaphoreType.DMA,
        pltpu.SemaphoreType.DMA,
        pltpu.SemaphoreType.DMA,
    ],
)
def _sc_scatter(m_hbm, dst_hbm, zeros_hbm, out,
                X_sh, rowsA, rowsB, dstbigA, dstbigB, dstvA, dstvB,
                lsemA, lsemB, ssemA, ssemB):
    cid = lax.axis_index("c")
    sid = lax.axis_index("s")
    lo = cid * NH
    bufs = ((rowsA, dstbigA, dstvA, lsemA, ssemA),
            (rowsB, dstbigB, dstvB, lsemB, ssemB))

    @pl.when(sid == 0)
    def _():
        pltpu.sync_copy(zeros_hbm.at[pl.ds(0, NHP)], X_sh)

    plsc.subcore_barrier()

    base0 = sid * ET
    pltpu.async_copy(m_hbm.at[pl.ds(base0, CBS)], rowsA, lsemA)
    pltpu.async_copy(dst_hbm.at[pl.ds(base0, CBS)], dstbigA, lsemA)

    def chunk(c, b_):
        rows, dstbig, dstv, lsem, ssem = bufs[b_]
        rowsN, dstbigN, dstvN, lsemN, ssemN = bufs[1 - b_]

        @pl.when(c >= 1)
        def _():
            for j in range(NSS):
                pltpu.make_async_copy(
                    rowsN.at[pl.ds(j * SUB, SUB)],
                    X_sh.at[dstvN.at[j]], ssemN).wait()

        @pl.when(c + 1 < NITS)
        def _():
            nbase = sid * ET + (c + 1) * CBS
            pltpu.async_copy(m_hbm.at[pl.ds(nbase, CBS)], rowsN, lsemN)
            pltpu.async_copy(dst_hbm.at[pl.ds(nbase, CBS)], dstbigN, lsemN)

        pltpu.make_async_copy(m_hbm.at[pl.ds(0, CBS)], rows, lsem).wait()
        pltpu.make_async_copy(dst_hbm.at[pl.ds(0, CBS)], dstbig, lsem).wait()
        for k in range(CBS // 16):
            t = dstbig[pl.ds(k * 16, 16)] - lo
            t = jnp.where((t >= 0) & (t < NH), t, NH)
            dstv[k // (SUB // 16), pl.ds((k % (SUB // 16)) * 16, 16)] = t
        for j in range(NSS):
            pltpu.async_copy(rows.at[pl.ds(j * SUB, SUB)],
                             X_sh.at[dstv.at[j]], ssem, add=True)

    def body(c, carry):
        for b_ in (0, 1):
            @pl.when(lax.rem(c, 2) == b_)
            def _(b_=b_):
                chunk(c, b_)
        return carry

    lax.fori_loop(0, NITS, body, 0)
    rowsL, dstbigL, dstvL, lsemL, ssemL = bufs[(NITS - 1) % 2]
    for j in range(NSS):
        pltpu.make_async_copy(rowsL.at[pl.ds(j * SUB, SUB)],
                              X_sh.at[dstvL.at[j]], ssemL).wait()
    plsc.subcore_barrier()

    @pl.when(sid == 0)
    def _():
        pltpu.sync_copy(X_sh.at[pl.ds(0, NH)], out.at[pl.ds(lo, NH)])


@functools.partial(
    pl.kernel,
    out_type=(
        jax.ShapeDtypeStruct((E,), f32),
        jax.ShapeDtypeStruct((E, D), f32),
    ),
    mesh=_mesh,
    compiler_params=pltpu.CompilerParams(needs_layout_passes=False),
    scratch_types=[
        pltpu.VMEM_SHARED((N,), f32),
        pltpu.VMEM((N,), f32),
        pltpu.VMEM((N,), f32),
        pltpu.VMEM((N,), f32),
        pltpu.VMEM((CB,), f32),
        pltpu.VMEM((CB,), i32),
        pltpu.VMEM((CB,), i32),
        pltpu.VMEM((NSUB, SUB), i32),
        pltpu.VMEM((NSUB, SUB), i32),
        pltpu.VMEM((CB, D), f32),
        pltpu.VMEM((CB,), f32),
        pltpu.SemaphoreType.DMA,
    ],
)
def _sc_edge1a(p1_hbm, p2_hbm, src_hbm, dst_hbm, x1_hbm, zeros_hbm,
               ssrc_out, rows_out,
               S_sh, p1t, p2t, stile, wv, srcbig, dstbig, srcv, dstv, rows,
               sbuf, sem):
    cid = lax.axis_index("c")
    sid = lax.axis_index("s")
    wid = _wid()
    pltpu.sync_copy(p1_hbm, p1t)
    pltpu.sync_copy(p2_hbm, p2t)

    @pl.when(sid == 0)
    def _():
        pltpu.sync_copy(zeros_hbm, S_sh)

    plsc.subcore_barrier()

    def ph1(i, carry):
        base = sid * ET + i * CB
        pltpu.sync_copy(src_hbm.at[pl.ds(base, CB)], srcbig)
        pltpu.sync_copy(dst_hbm.at[pl.ds(base, CB)], dstbig)
        _stage_idx(srcbig, srcv)
        for k in range(CB // 16):
            idxd = dstbig[pl.ds(k * 16, 16)]
            idxs = srcbig[pl.ds(k * 16, 16)]
            t = plsc.load_gather(p1t, [idxd]) + plsc.load_gather(p2t, [idxs])
            t = jnp.where(t >= 0.0, t, 0.01 * t)
            wv[pl.ds(k * 16, 16)] = jnp.exp(t)
        descs = []
        for j in range(NSUB):
            descs.append(pltpu.async_copy(
                wv.at[pl.ds(j * SUB, SUB)], S_sh.at[srcv.at[j]], sem,
                add=True))
        for d in descs:
            d.wait()
        return carry

    lax.fori_loop(0, ET // CB, ph1, 0)
    plsc.subcore_barrier()
    pltpu.sync_copy(S_sh, stile)

    def ph2(i, carry):
        base = wid * EW + i * CB
        pltpu.sync_copy(src_hbm.at[pl.ds(base, CB)], srcbig)
        pltpu.sync_copy(dst_hbm.at[pl.ds(base, CB)], dstbig)
        _stage_idx(dstbig, dstv)
        descs = []
        for j in range(NSUB):
            descs.append(pltpu.async_copy(
                x1_hbm.at[dstv.at[j]], rows.at[pl.ds(j * SUB, SUB)], sem))
        for k in range(CB // 16):
            idxd = dstbig[pl.ds(k * 16, 16)]
            idxs = srcbig[pl.ds(k * 16, 16)]
            t = plsc.load_gather(p1t, [idxd]) + plsc.load_gather(p2t, [idxs])
            t = jnp.where(t >= 0.0, t, 0.01 * t)
            sv = plsc.load_gather(stile, [idxs])
            sbuf[pl.ds(k * 16, 16)] = jnp.exp(t) / (sv + 1e-16)
        for d in descs:
            d.wait()
        pltpu.sync_copy(rows, rows_out.at[pl.ds(base, CB)])
        pltpu.sync_copy(sbuf, ssrc_out.at[pl.ds(base, CB)])
        return carry

    lax.fori_loop(0, EW // CB, ph2, 0)


def _leaky(t):
    return jnp.where(t >= 0.0, t, 0.01 * t)


def _dot(a, b):
    return jnp.dot(a, b, preferred_element_type=f32)


def _tc_prep0_body(x_ref, wa_ref, ba_ref, wn_ref, wal_ref, u_ref, v_ref, p_ref):
    x = x_ref[...]
    u = _leaky(_dot(x, wa_ref[...]) + ba_ref[...])
    u_ref[...] = u
    v_ref[...] = _dot(x, wn_ref[...])
    p_ref[...] = _dot(u, wal_ref[...])


def _tc_prep0(x, W_atom0, b_atom0, Wnb1, Wal0a):
    return pl.pallas_call(
        _tc_prep0_body,
        out_shape=(
            jax.ShapeDtypeStruct((N, D), f32),
            jax.ShapeDtypeStruct((N, D), f32),
            jax.ShapeDtypeStruct((N, 1), f32),
        ),
    )(x, W_atom0, b_atom0.reshape(1, D), Wnb1, Wal0a)


BE = 3200
BEH = BE // 128


def _tc_edge0a_body(vs_ref, ea_ref, pd_ref, wn2_ref, bn_ref, wb_ref, bal_ref,
                    w_ref):
    ea = lax.dot_general(ea_ref[...], wn2_ref[...], (((0,), (0,)), ((), ())),
                         preferred_element_type=f32)
    xj = _leaky(vs_ref[...] + ea + bn_ref[...])
    ej = _dot(xj, wb_ref[...])
    pdt = jnp.transpose(pd_ref[0])
    pdcol = jnp.concatenate([pdt[:, r:r + 1] for r in range(BEH)], axis=0)
    e = _leaky(pdcol + ej + bal_ref[...])
    w = jnp.exp(e)
    rows = [jnp.transpose(w[r * 128:(r + 1) * 128, :]) for r in range(BEH)]
    w_ref[...] = jnp.reshape(jnp.concatenate(rows, axis=0), (1, BEH, 128))


def _make_tc_edge0a(eoff, esz):
    def run(Vsrc, eaT, Pdst, Wnb2, b_nb0, Wb0, b_align0):
        full = lambda sh: pl.BlockSpec(sh, lambda i: (0, 0))
        boff = eoff // BE
        return pl.pallas_call(
            _tc_edge0a_body,
            grid=(esz // BE,),
            compiler_params=pltpu.CompilerParams(
                fuse_transposed_lhs_in_matmul=True),
            in_specs=[
                pl.BlockSpec((BE, D), lambda i: (i, 0)),
                pl.BlockSpec((16, BE), lambda i: (0, i + boff)),
                pl.BlockSpec((1, BEH, 128), lambda i: (i, 0, 0)),
                full((16, D)),
                full((1, D)),
                full((D, 1)),
                full((1, 1)),
            ],
            out_specs=pl.BlockSpec((1, BEH, 128), lambda i: (i, 0, 0)),
            out_shape=jax.ShapeDtypeStruct((esz // BE, BEH, 128), f32),
        )(Vsrc, eaT, Pdst, Wnb2, b_nb0.reshape(1, D), Wb0,
          b_align0.reshape(1, 1))
    return run


_tc_edge0a_h = (_make_tc_edge0a(0, H0), _make_tc_edge0a(H0, H1))


def _tc_edge_msg_body(xi_ref, a_ref, watt_ref, batt_ref,
                      wih_ref, bih_ref, whh_ref, bhh_ref, m_ref):
    x_i = xi_ref[...]
    at = jnp.transpose(a_ref[0])
    q = _dot(x_i, watt_ref[...]) + batt_ref[...]
    aq = jnp.concatenate(
        [q[r * 128:(r + 1) * 128, :] * at[:, r:r + 1] for r in range(BEH)],
        axis=0)
    c = jnp.where(aq > 0.0, aq, jnp.exp(aq) - 1.0)
    gi = _dot(c, wih_ref[...]) + bih_ref[...]
    gh = _dot(x_i, whh_ref[...]) + bhh_ref[...]
    r = jax.nn.sigmoid(gi[:, :D] + gh[:, :D])
    z = jax.nn.sigmoid(gi[:, D:2 * D] + gh[:, D:2 * D])
    n = jnp.tanh(gi[:, 2 * D:] + r * gh[:, 2 * D:])
    m_ref[...] = (1.0 - z) * n + z * x_i


def _tc_edge_msg(Xd, a, W_att, b_att, WihT, bih, WhhT, bhh):
    full = lambda s: pl.BlockSpec(s, lambda i: (0, 0))
    return pl.pallas_call(
        _tc_edge_msg_body,
        grid=(E // BE,),
        in_specs=[
            pl.BlockSpec((BE, D), lambda i: (i, 0)),
            pl.BlockSpec((1, BEH, 128), lambda i: (i, 0, 0)),
            full((D, D)),
            full((1, D)),
            full((D, 3 * D)),
            full((1, 3 * D)),
            full((D, 3 * D)),
            full((1, 3 * D)),
        ],
        out_specs=pl.BlockSpec((BE, D), lambda i: (i, 0)),
        out_shape=jax.ShapeDtypeStruct((E, D), f32),
    )(Xd, a, W_att, b_att.reshape(1, D), WihT, bih.reshape(1, 3 * D),
      WhhT, bhh.reshape(1, 3 * D))


def _tc_prep1_body(x1_ref, wa_ref, wb_ref, bal_ref, p1_ref, p2_ref):
    x1 = x1_ref[...]
    p1_ref[...] = _dot(x1, wa_ref[...]) + bal_ref[...]
    p2_ref[...] = _dot(x1, wb_ref[...])


def _tc_prep1(x1, Wa1, Wb1, b_align1):
    return pl.pallas_call(
        _tc_prep1_body,
        out_shape=(
            jax.ShapeDtypeStruct((N, 1), f32),
            jax.ShapeDtypeStruct((N, 1), f32),
        ),
    )(x1, Wa1, Wb1, b_align1.reshape(1, 1))


def kernel(x, edge_index, edge_attr,
           W_atom0, b_atom0, W_nb0, b_nb0, W_align0, b_align0, W_att0, b_att0,
           Wih0, Whh0, bih0, bhh0,
           W_align1, b_align1, W_att1, b_att1, Wih1, Whh1, bih1, bhh1):
    src = edge_index[0]
    dst = edge_index[1]
    zeros_n = jnp.zeros((N,), f32)
    zeros_nd = jnp.zeros((N, D), f32)

    u, v, p = _tc_prep0(x, W_atom0, b_atom0, W_nb0[:D], W_align0[:D])
    pflat = p.reshape(N)
    eaT = edge_attr.T
    Vs0, Pd0 = _sc_gather0_h[0](v, pflat, src, dst)
    Vs1, Pd1 = _sc_gather0_h[1](v, pflat, src, dst)
    w0 = _tc_edge0a_h[0](Vs0, eaT, Pd0.reshape(H0 // BE, BEH, 128),
                         W_nb0[D:], b_nb0, W_align0[D:], b_align0)
    w1 = _tc_edge0a_h[1](Vs1, eaT, Pd1.reshape(H1 // BE, BEH, 128),
                         W_nb0[D:], b_nb0, W_align0[D:], b_align0)
    Adst, Udst = _sc_seg_gather(w0.reshape(H0), w1.reshape(H1), src, dst, u,
                                zeros_n)
    M = _tc_edge_msg(Udst, Adst.reshape(E // BE, BEH, 128), W_att0, b_att0,
                     Wih0.T, bih0, Whh0.T, bhh0)
    x1 = _sc_scatter(M, dst, zeros_nd)

    p1, p2 = _tc_prep1(x1, W_align1[:D], W_align1[D:], b_align1)
    A1, X1dst = _sc_edge1a(p1.reshape(N), p2.reshape(N), src, dst, x1, zeros_n)
    M1 = _tc_edge_msg(X1dst, A1.reshape(E // BE, BEH, 128), W_att1, b_att1,
                      Wih1.T, bih1, Whh1.T, bhh1)
    return _sc_scatter(M1, dst, zeros_nd)


# speedup vs baseline: 9.0585x; 1.1252x over previous
import functools

import jax
import jax.numpy as jnp
from jax import lax
from jax.experimental import pallas as pl
from jax.experimental.pallas import tpu as pltpu
from jax.experimental.pallas import tpu_sc as plsc

N = 10000
E = 320000
D = 128
NC = 2
NS = 16
NW = NC * NS
EW = E // NW
ET = E // NS
SUB = 80
CB = 400
NSUB = CB // SUB

_mesh = plsc.VectorSubcoreMesh(core_axis_name="c", subcore_axis_name="s")

f32 = jnp.float32
i32 = jnp.int32


def _wid():
    return lax.axis_index("s") * NC + lax.axis_index("c")


def _stage_idx(big, two_d):
    for j in range(NSUB):
        for k in range(SUB // 16):
            two_d[j, pl.ds(k * 16, 16)] = big[pl.ds(j * SUB + k * 16, 16)]


H0 = 153600
H1 = E - H0


def _make_gather0(eoff, esz):
    ewh = esz // NW

    @functools.partial(
        pl.kernel,
        out_type=(
            jax.ShapeDtypeStruct((esz, D), f32),
            jax.ShapeDtypeStruct((esz,), f32),
        ),
        mesh=_mesh,
        compiler_params=pltpu.CompilerParams(needs_layout_passes=False),
        scratch_types=[
            pltpu.VMEM((N,), f32),
            pltpu.VMEM((CB,), i32),
            pltpu.VMEM((CB,), i32),
            pltpu.VMEM((NSUB, SUB), i32),
            pltpu.VMEM((CB, D), f32),
            pltpu.VMEM((CB,), f32),
            pltpu.SemaphoreType.DMA,
        ],
    )
    def k(v_hbm, p_hbm, src_hbm, dst_hbm, vout, pout,
          ptab, srcbig, dstbig, srcv, rows, pbuf, sem):
        wid = _wid()
        pltpu.sync_copy(p_hbm, ptab)

        def outer(i, carry):
            lbase = wid * ewh + i * CB
            gbase = eoff + lbase
            pltpu.sync_copy(src_hbm.at[pl.ds(gbase, CB)], srcbig)
            pltpu.sync_copy(dst_hbm.at[pl.ds(gbase, CB)], dstbig)
            _stage_idx(srcbig, srcv)
            descs = []
            for j in range(NSUB):
                descs.append(pltpu.async_copy(
                    v_hbm.at[srcv.at[j]], rows.at[pl.ds(j * SUB, SUB)], sem))
            for kk in range(CB // 16):
                idx = dstbig[pl.ds(kk * 16, 16)]
                pbuf[pl.ds(kk * 16, 16)] = plsc.load_gather(ptab, [idx])
            for d in descs:
                d.wait()
            pltpu.sync_copy(rows, vout.at[pl.ds(lbase, CB)])
            pltpu.sync_copy(pbuf, pout.at[pl.ds(lbase, CB)])
            return carry

        lax.fori_loop(0, ewh // CB, outer, 0)

    return k


_sc_gather0_h = (_make_gather0(0, H0), _make_gather0(H0, H1))


@functools.partial(
    pl.kernel,
    out_type=(
        jax.ShapeDtypeStruct((H0,), f32),
        jax.ShapeDtypeStruct((H1,), f32),
        jax.ShapeDtypeStruct((H0, D), f32),
        jax.ShapeDtypeStruct((H1, D), f32),
    ),
    mesh=_mesh,
    compiler_params=pltpu.CompilerParams(needs_layout_passes=False),
    scratch_types=[
        pltpu.VMEM_SHARED((N,), f32),
        pltpu.VMEM((N,), f32),
        pltpu.VMEM((CB,), f32),
        pltpu.VMEM((CB,), i32),
        pltpu.VMEM((CB,), i32),
        pltpu.VMEM((NSUB, SUB), i32),
        pltpu.VMEM((NSUB, SUB), i32),
        pltpu.VMEM((CB, D), f32),
        pltpu.VMEM((CB,), f32),
        pltpu.SemaphoreType.DMA,
    ],
)
def _sc_seg_gather(w0_hbm, w1_hbm, src_hbm, dst_hbm, tab_hbm, zeros_hbm,
                   a0_out, a1_out, r0_out, r1_out,
                   S_sh, stile, wv, srcbig, dstbig, srcv, dstv, rows, sbuf,
                   sem):
    sid = lax.axis_index("s")
    wid = _wid()

    @pl.when(sid == 0)
    def _():
        pltpu.sync_copy(zeros_hbm, S_sh)

    plsc.subcore_barrier()

    def ph1a(i, carry):
        lbase = sid * (H0 // NS) + i * CB
        pltpu.sync_copy(w0_hbm.at[pl.ds(lbase, CB)], wv)
        pltpu.sync_copy(src_hbm.at[pl.ds(lbase, CB)], srcbig)
        _stage_idx(srcbig, srcv)
        descs = []
        for j in range(NSUB):
            descs.append(pltpu.async_copy(
                wv.at[pl.ds(j * SUB, SUB)], S_sh.at[srcv.at[j]], sem,
                add=True))
        for d in descs:
            d.wait()
        return carry

    def ph1b(i, carry):
        lbase = sid * (H1 // NS) + i * CB
        pltpu.sync_copy(w1_hbm.at[pl.ds(lbase, CB)], wv)
        pltpu.sync_copy(src_hbm.at[pl.ds(H0 + lbase, CB)], srcbig)
        _stage_idx(srcbig, srcv)
        descs = []
        for j in range(NSUB):
            descs.append(pltpu.async_copy(
                wv.at[pl.ds(j * SUB, SUB)], S_sh.at[srcv.at[j]], sem,
                add=True))
        for d in descs:
            d.wait()
        return carry

    lax.fori_loop(0, H0 // NS // CB, ph1a, 0)
    lax.fori_loop(0, H1 // NS // CB, ph1b, 0)
    plsc.subcore_barrier()
    pltpu.sync_copy(S_sh, stile)

    def ph2(i, carry):
        base = wid * EW + i * CB
        pltpu.sync_copy(src_hbm.at[pl.ds(base, CB)], srcbig)
        pltpu.sync_copy(dst_hbm.at[pl.ds(base, CB)], dstbig)

        @pl.when(base < H0)
        def _():
            pltpu.sync_copy(w0_hbm.at[pl.ds(base, CB)], wv)

        @pl.when(base >= H0)
        def _():
            pltpu.sync_copy(w1_hbm.at[pl.ds(base - H0, CB)], wv)

        _stage_idx(dstbig, dstv)
        descs = []
        for j in range(NSUB):
            descs.append(pltpu.async_copy(
                tab_hbm.at[dstv.at[j]], rows.at[pl.ds(j * SUB, SUB)], sem))
        for k in range(CB // 16):
            idx = srcbig[pl.ds(k * 16, 16)]
            sv = plsc.load_gather(stile, [idx])
            sbuf[pl.ds(k * 16, 16)] = wv[pl.ds(k * 16, 16)] / (sv + 1e-16)
        for d in descs:
            d.wait()

        @pl.when(base < H0)
        def _():
            pltpu.sync_copy(rows, r0_out.at[pl.ds(base, CB)])
            pltpu.sync_copy(sbuf, a0_out.at[pl.ds(base, CB)])

        @pl.when(base >= H0)
        def _():
            pltpu.sync_copy(rows, r1_out.at[pl.ds(base - H0, CB)])
            pltpu.sync_copy(sbuf, a1_out.at[pl.ds(base - H0, CB)])

        return carry

    lax.fori_loop(0, EW // CB, ph2, 0)


NH = N // 2
NHP = NH + 8
CBS = 160
NSS = CBS // SUB


def _make_scatter(eoff, esz):
    nits = esz // NS // CBS
    tsz = esz // NS

    @functools.partial(
        pl.kernel,
        out_type=jax.ShapeDtypeStruct((N, D), f32),
        mesh=_mesh,
        compiler_params=pltpu.CompilerParams(needs_layout_passes=False),
        scratch_types=[
            pltpu.VMEM_SHARED((NHP, D), f32),
            pltpu.VMEM((CBS, D), f32),
            pltpu.VMEM((CBS, D), f32),
            pltpu.VMEM((CBS,), i32),
            pltpu.VMEM((CBS,), i32),
            pltpu.VMEM((NSS, SUB), i32),
            pltpu.VMEM((NSS, SUB), i32),
            pltpu.SemaphoreType.DMA,
            pltpu.SemaphoreType.DMA,
            pltpu.SemaphoreType.DMA,
            pltpu.SemaphoreType.DMA,
        ],
    )
    def k(m_hbm, dst_hbm, zeros_hbm, out,
          X_sh, rowsA, rowsB, dstbigA, dstbigB, dstvA, dstvB,
          lsemA, lsemB, ssemA, ssemB):
        cid = lax.axis_index("c")
        sid = lax.axis_index("s")
        lo = cid * NH
        bufs = ((rowsA, dstbigA, dstvA, lsemA, ssemA),
                (rowsB, dstbigB, dstvB, lsemB, ssemB))

        @pl.when(sid == 0)
        def _():
            pltpu.sync_copy(zeros_hbm.at[pl.ds(0, NHP)], X_sh)

        plsc.subcore_barrier()

        base0 = sid * tsz
        pltpu.async_copy(m_hbm.at[pl.ds(base0, CBS)], rowsA, lsemA)
        pltpu.async_copy(dst_hbm.at[pl.ds(eoff + base0, CBS)], dstbigA, lsemA)

        def chunk(c, b_):
            rows, dstbig, dstv, lsem, ssem = bufs[b_]
            rowsN, dstbigN, dstvN, lsemN, ssemN = bufs[1 - b_]

            @pl.when(c >= 1)
            def _():
                for j in range(NSS):
                    pltpu.make_async_copy(
                        rowsN.at[pl.ds(j * SUB, SUB)],
                        X_sh.at[dstvN.at[j]], ssemN).wait()

            @pl.when(c + 1 < nits)
            def _():
                nbase = sid * tsz + (c + 1) * CBS
                pltpu.async_copy(m_hbm.at[pl.ds(nbase, CBS)], rowsN, lsemN)
                pltpu.async_copy(dst_hbm.at[pl.ds(eoff + nbase, CBS)],
                                 dstbigN, lsemN)

            pltpu.make_async_copy(m_hbm.at[pl.ds(0, CBS)], rows, lsem).wait()
            pltpu.make_async_copy(dst_hbm.at[pl.ds(0, CBS)], dstbig,
                                  lsem).wait()
            for kk in range(CBS // 16):
                t = dstbig[pl.ds(kk * 16, 16)] - lo
                t = jnp.where((t >= 0) & (t < NH), t, NH)
                dstv[kk // (SUB // 16),
                     pl.ds((kk % (SUB // 16)) * 16, 16)] = t
            for j in range(NSS):
                pltpu.async_copy(rows.at[pl.ds(j * SUB, SUB)],
                                 X_sh.at[dstv.at[j]], ssem, add=True)

        def body(c, carry):
            for b_ in (0, 1):
                @pl.when(lax.rem(c, 2) == b_)
                def _(b_=b_):
                    chunk(c, b_)
            return carry

        lax.fori_loop(0, nits, body, 0)
        rowsL, dstbigL, dstvL, lsemL, ssemL = bufs[(nits - 1) % 2]
        for j in range(NSS):
            pltpu.make_async_copy(rowsL.at[pl.ds(j * SUB, SUB)],
                                  X_sh.at[dstvL.at[j]], ssemL).wait()
        plsc.subcore_barrier()

        @pl.when(sid == 0)
        def _():
            pltpu.sync_copy(X_sh.at[pl.ds(0, NH)], out.at[pl.ds(lo, NH)])

    return k


_sc_scatter_h = (_make_scatter(0, H0), _make_scatter(H0, H1))


@functools.partial(
    pl.kernel,
    out_type=(
        jax.ShapeDtypeStruct((H0,), f32),
        jax.ShapeDtypeStruct((H1,), f32),
        jax.ShapeDtypeStruct((H0, D), f32),
        jax.ShapeDtypeStruct((H1, D), f32),
    ),
    mesh=_mesh,
    compiler_params=pltpu.CompilerParams(needs_layout_passes=False),
    scratch_types=[
        pltpu.VMEM_SHARED((N,), f32),
        pltpu.VMEM((N,), f32),
        pltpu.VMEM((N,), f32),
        pltpu.VMEM((N,), f32),
        pltpu.VMEM((CB,), f32),
        pltpu.VMEM((CB,), i32),
        pltpu.VMEM((CB,), i32),
        pltpu.VMEM((NSUB, SUB), i32),
        pltpu.VMEM((NSUB, SUB), i32),
        pltpu.VMEM((CB, D), f32),
        pltpu.VMEM((CB,), f32),
        pltpu.SemaphoreType.DMA,
    ],
)
def _sc_edge1a(p1_hbm, p2_hbm, src_hbm, dst_hbm, x1_hbm, zeros_hbm,
               a0_out, a1_out, r0_out, r1_out,
               S_sh, p1t, p2t, stile, wv, srcbig, dstbig, srcv, dstv, rows,
               sbuf, sem):
    cid = lax.axis_index("c")
    sid = lax.axis_index("s")
    wid = _wid()
    pltpu.sync_copy(p1_hbm, p1t)
    pltpu.sync_copy(p2_hbm, p2t)

    @pl.when(sid == 0)
    def _():
        pltpu.sync_copy(zeros_hbm, S_sh)

    plsc.subcore_barrier()

    def ph1(i, carry):
        base = sid * ET + i * CB
        pltpu.sync_copy(src_hbm.at[pl.ds(base, CB)], srcbig)
        pltpu.sync_copy(dst_hbm.at[pl.ds(base, CB)], dstbig)
        _stage_idx(srcbig, srcv)
        for k in range(CB // 16):
            idxd = dstbig[pl.ds(k * 16, 16)]
            idxs = srcbig[pl.ds(k * 16, 16)]
            t = plsc.load_gather(p1t, [idxd]) + plsc.load_gather(p2t, [idxs])
            t = jnp.where(t >= 0.0, t, 0.01 * t)
            wv[pl.ds(k * 16, 16)] = jnp.exp(t)
        descs = []
        for j in range(NSUB):
            descs.append(pltpu.async_copy(
                wv.at[pl.ds(j * SUB, SUB)], S_sh.at[srcv.at[j]], sem,
                add=True))
        for d in descs:
            d.wait()
        return carry

    lax.fori_loop(0, ET // CB, ph1, 0)
    plsc.subcore_barrier()
    pltpu.sync_copy(S_sh, stile)

    def ph2(i, carry):
        base = wid * EW + i * CB
        pltpu.sync_copy(src_hbm.at[pl.ds(base, CB)], srcbig)
        pltpu.sync_copy(dst_hbm.at[pl.ds(base, CB)], dstbig)
        _stage_idx(dstbig, dstv)
        descs = []
        for j in range(NSUB):
            descs.append(pltpu.async_copy(
                x1_hbm.at[dstv.at[j]], rows.at[pl.ds(j * SUB, SUB)], sem))
        for k in range(CB // 16):
            idxd = dstbig[pl.ds(k * 16, 16)]
            idxs = srcbig[pl.ds(k * 16, 16)]
            t = plsc.load_gather(p1t, [idxd]) + plsc.load_gather(p2t, [idxs])
            t = jnp.where(t >= 0.0, t, 0.01 * t)
            sv = plsc.load_gather(stile, [idxs])
            sbuf[pl.ds(k * 16, 16)] = jnp.exp(t) / (sv + 1e-16)
        for d in descs:
            d.wait()
        @pl.when(base < H0)
        def _():
            pltpu.sync_copy(rows, r0_out.at[pl.ds(base, CB)])
            pltpu.sync_copy(sbuf, a0_out.at[pl.ds(base, CB)])

        @pl.when(base >= H0)
        def _():
            pltpu.sync_copy(rows, r1_out.at[pl.ds(base - H0, CB)])
            pltpu.sync_copy(sbuf, a1_out.at[pl.ds(base - H0, CB)])

        return carry

    lax.fori_loop(0, EW // CB, ph2, 0)


def _leaky(t):
    return jnp.where(t >= 0.0, t, 0.01 * t)


def _dot(a, b):
    return jnp.dot(a, b, preferred_element_type=f32)


def _tc_prep0_body(x_ref, wa_ref, ba_ref, wn_ref, wal_ref, u_ref, v_ref, p_ref):
    x = x_ref[...]
    u = _leaky(_dot(x, wa_ref[...]) + ba_ref[...])
    u_ref[...] = u
    v_ref[...] = _dot(x, wn_ref[...])
    p_ref[...] = _dot(u, wal_ref[...])


def _tc_prep0(x, W_atom0, b_atom0, Wnb1, Wal0a):
    return pl.pallas_call(
        _tc_prep0_body,
        out_shape=(
            jax.ShapeDtypeStruct((N, D), f32),
            jax.ShapeDtypeStruct((N, D), f32),
            jax.ShapeDtypeStruct((N, 1), f32),
        ),
    )(x, W_atom0, b_atom0.reshape(1, D), Wnb1, Wal0a)


BE = 3200
BEH = BE // 128


def _tc_edge0a_body(vs_ref, ea_ref, pd_ref, wn2_ref, bn_ref, wb_ref, bal_ref,
                    w_ref):
    ea = lax.dot_general(ea_ref[...], wn2_ref[...], (((0,), (0,)), ((), ())),
                         preferred_element_type=f32)
    xj = _leaky(vs_ref[...] + ea + bn_ref[...])
    ej = _dot(xj, wb_ref[...])
    pdt = jnp.transpose(pd_ref[0])
    pdcol = jnp.concatenate([pdt[:, r:r + 1] for r in range(BEH)], axis=0)
    e = _leaky(pdcol + ej + bal_ref[...])
    w = jnp.exp(e)
    rows = [jnp.transpose(w[r * 128:(r + 1) * 128, :]) for r in range(BEH)]
    w_ref[...] = jnp.reshape(jnp.concatenate(rows, axis=0), (1, BEH, 128))


def _make_tc_edge0a(eoff, esz):
    def run(Vsrc, eaT, Pdst, Wnb2, b_nb0, Wb0, b_align0):
        full = lambda sh: pl.BlockSpec(sh, lambda i: (0, 0))
        boff = eoff // BE
        return pl.pallas_call(
            _tc_edge0a_body,
            grid=(esz // BE,),
            compiler_params=pltpu.CompilerParams(
                fuse_transposed_lhs_in_matmul=True),
            in_specs=[
                pl.BlockSpec((BE, D), lambda i: (i, 0)),
                pl.BlockSpec((16, BE), lambda i: (0, i + boff)),
                pl.BlockSpec((1, BEH, 128), lambda i: (i, 0, 0)),
                full((16, D)),
                full((1, D)),
                full((D, 1)),
                full((1, 1)),
            ],
            out_specs=pl.BlockSpec((1, BEH, 128), lambda i: (i, 0, 0)),
            out_shape=jax.ShapeDtypeStruct((esz // BE, BEH, 128), f32),
        )(Vsrc, eaT, Pdst, Wnb2, b_nb0.reshape(1, D), Wb0,
          b_align0.reshape(1, 1))
    return run


_tc_edge0a_h = (_make_tc_edge0a(0, H0), _make_tc_edge0a(H0, H1))


def _tc_edge_msg_body(xi_ref, a_ref, watt_ref, batt_ref,
                      wih_ref, bih_ref, whh_ref, bhh_ref, m_ref):
    x_i = xi_ref[...]
    at = jnp.transpose(a_ref[0])
    q = _dot(x_i, watt_ref[...]) + batt_ref[...]
    aq = jnp.concatenate(
        [q[r * 128:(r + 1) * 128, :] * at[:, r:r + 1] for r in range(BEH)],
        axis=0)
    c = jnp.where(aq > 0.0, aq, jnp.exp(aq) - 1.0)
    gi = _dot(c, wih_ref[...]) + bih_ref[...]
    gh = _dot(x_i, whh_ref[...]) + bhh_ref[...]
    r = jax.nn.sigmoid(gi[:, :D] + gh[:, :D])
    z = jax.nn.sigmoid(gi[:, D:2 * D] + gh[:, D:2 * D])
    n = jnp.tanh(gi[:, 2 * D:] + r * gh[:, 2 * D:])
    m_ref[...] = (1.0 - z) * n + z * x_i


def _make_tc_edge_msg(esz):
    def run(Xd, a, W_att, b_att, WihT, bih, WhhT, bhh):
        full = lambda sh: pl.BlockSpec(sh, lambda i: (0, 0))
        return pl.pallas_call(
            _tc_edge_msg_body,
            grid=(esz // BE,),
            in_specs=[
                pl.BlockSpec((BE, D), lambda i: (i, 0)),
                pl.BlockSpec((1, BEH, 128), lambda i: (i, 0, 0)),
                full((D, D)),
                full((1, D)),
                full((D, 3 * D)),
                full((1, 3 * D)),
                full((D, 3 * D)),
                full((1, 3 * D)),
            ],
            out_specs=pl.BlockSpec((BE, D), lambda i: (i, 0)),
            out_shape=jax.ShapeDtypeStruct((esz, D), f32),
        )(Xd, a, W_att, b_att.reshape(1, D), WihT, bih.reshape(1, 3 * D),
          WhhT, bhh.reshape(1, 3 * D))
    return run


_tc_edge_msg_h = (_make_tc_edge_msg(H0), _make_tc_edge_msg(H1))


def _tc_prep1_body(xa_ref, xb_ref, wa_ref, wb_ref, bal_ref,
                   x1_ref, p1_ref, p2_ref):
    x1 = xa_ref[...] + xb_ref[...]
    x1_ref[...] = x1
    p1_ref[...] = _dot(x1, wa_ref[...]) + bal_ref[...]
    p2_ref[...] = _dot(x1, wb_ref[...])


def _tc_prep1(xa, xb, Wa1, Wb1, b_align1):
    return pl.pallas_call(
        _tc_prep1_body,
        out_shape=(
            jax.ShapeDtypeStruct((N, D), f32),
            jax.ShapeDtypeStruct((N, 1), f32),
            jax.ShapeDtypeStruct((N, 1), f32),
        ),
    )(xa, xb, Wa1, Wb1, b_align1.reshape(1, 1))


def _tc_final_body(xa_ref, xb_ref, out_ref):
    out_ref[...] = xa_ref[...] + xb_ref[...]


def _tc_final(xa, xb):
    return pl.pallas_call(
        _tc_final_body,
        out_shape=jax.ShapeDtypeStruct((N, D), f32),
    )(xa, xb)


def kernel(x, edge_index, edge_attr,
           W_atom0, b_atom0, W_nb0, b_nb0, W_align0, b_align0, W_att0, b_att0,
           Wih0, Whh0, bih0, bhh0,
           W_align1, b_align1, W_att1, b_att1, Wih1, Whh1, bih1, bhh1):
    src = edge_index[0]
    dst = edge_index[1]
    zeros_n = jnp.zeros((N,), f32)
    zeros_nd = jnp.zeros((N, D), f32)

    u, v, p = _tc_prep0(x, W_atom0, b_atom0, W_nb0[:D], W_align0[:D])
    pflat = p.reshape(N)
    eaT = edge_attr.T
    Vs0, Pd0 = _sc_gather0_h[0](v, pflat, src, dst)
    Vs1, Pd1 = _sc_gather0_h[1](v, pflat, src, dst)
    w0 = _tc_edge0a_h[0](Vs0, eaT, Pd0.reshape(H0 // BE, BEH, 128),
                         W_nb0[D:], b_nb0, W_align0[D:], b_align0)
    w1 = _tc_edge0a_h[1](Vs1, eaT, Pd1.reshape(H1 // BE, BEH, 128),
                         W_nb0[D:], b_nb0, W_align0[D:], b_align0)
    A0, A1, Ud0, Ud1 = _sc_seg_gather(w0.reshape(H0), w1.reshape(H1),
                                      src, dst, u, zeros_n)
    M0 = _tc_edge_msg_h[0](Ud0, A0.reshape(H0 // BE, BEH, 128),
                           W_att0, b_att0, Wih0.T, bih0, Whh0.T, bhh0)
    M1 = _tc_edge_msg_h[1](Ud1, A1.reshape(H1 // BE, BEH, 128),
                           W_att0, b_att0, Wih0.T, bih0, Whh0.T, bhh0)
    xp0 = _sc_scatter_h[0](M0, dst, zeros_nd)
    xp1 = _sc_scatter_h[1](M1, dst, zeros_nd)

    x1, p1, p2 = _tc_prep1(xp0, xp1, W_align1[:D], W_align1[D:], b_align1)
    B0, B1, Xd0, Xd1 = _sc_edge1a(p1.reshape(N), p2.reshape(N), src, dst,
                                  x1, zeros_n)
    N0 = _tc_edge_msg_h[0](Xd0, B0.reshape(H0 // BE, BEH, 128),
                           W_att1, b_att1, Wih1.T, bih1, Whh1.T, bhh1)
    N1 = _tc_edge_msg_h[1](Xd1, B1.reshape(H1 // BE, BEH, 128),
                           W_att1, b_att1, Wih1.T, bih1, Whh1.T, bhh1)
    yp0 = _sc_scatter_h[0](N0, dst, zeros_nd)
    yp1 = _sc_scatter_h[1](N1, dst, zeros_nd)
    return _tc_final(yp0, yp1)


# speedup vs baseline: 9.8563x; 1.0881x over previous
import functools

import jax
import jax.numpy as jnp
from jax import lax
from jax.experimental import pallas as pl
from jax.experimental.pallas import tpu as pltpu
from jax.experimental.pallas import tpu_sc as plsc

N = 10000
E = 320000
D = 128
NC = 2
NS = 16
NW = NC * NS
EW = E // NW
ET = E // NS
SUB = 80
CB = 400
NSUB = CB // SUB

_mesh = plsc.VectorSubcoreMesh(core_axis_name="c", subcore_axis_name="s")

f32 = jnp.float32
i32 = jnp.int32


def _wid():
    return lax.axis_index("s") * NC + lax.axis_index("c")


def _stage_idx(big, two_d):
    for j in range(NSUB):
        for k in range(SUB // 16):
            two_d[j, pl.ds(k * 16, 16)] = big[pl.ds(j * SUB + k * 16, 16)]


H0 = 153600
H1 = E - H0
EW0 = H0 // NW
EW1 = H1 // NW


def _make_gather0(eoff, esz):
    ewh = esz // NW

    @functools.partial(
        pl.kernel,
        out_type=(
            jax.ShapeDtypeStruct((esz, D), f32),
            jax.ShapeDtypeStruct((esz,), f32),
        ),
        mesh=_mesh,
        compiler_params=pltpu.CompilerParams(needs_layout_passes=False),
        scratch_types=[
            pltpu.VMEM((N,), f32),
            pltpu.VMEM((CB,), i32),
            pltpu.VMEM((CB,), i32),
            pltpu.VMEM((NSUB, SUB), i32),
            pltpu.VMEM((CB, D), f32),
            pltpu.VMEM((CB,), f32),
            pltpu.SemaphoreType.DMA,
        ],
    )
    def k(v_hbm, p_hbm, src_hbm, dst_hbm, vout, pout,
          ptab, srcbig, dstbig, srcv, rows, pbuf, sem):
        wid = _wid()
        pltpu.sync_copy(p_hbm, ptab)

        def outer(i, carry):
            lbase = wid * ewh + i * CB
            gbase = eoff + lbase
            pltpu.sync_copy(src_hbm.at[pl.ds(gbase, CB)], srcbig)
            pltpu.sync_copy(dst_hbm.at[pl.ds(gbase, CB)], dstbig)
            _stage_idx(srcbig, srcv)
            descs = []
            for j in range(NSUB):
                descs.append(pltpu.async_copy(
                    v_hbm.at[srcv.at[j]], rows.at[pl.ds(j * SUB, SUB)], sem))
            for kk in range(CB // 16):
                idx = dstbig[pl.ds(kk * 16, 16)]
                pbuf[pl.ds(kk * 16, 16)] = plsc.load_gather(ptab, [idx])
            for d in descs:
                d.wait()
            pltpu.sync_copy(rows, vout.at[pl.ds(lbase, CB)])
            pltpu.sync_copy(pbuf, pout.at[pl.ds(lbase, CB)])
            return carry

        lax.fori_loop(0, ewh // CB, outer, 0)

    return k


_sc_gather0_h = (_make_gather0(0, H0), _make_gather0(H0, H1))


@functools.partial(
    pl.kernel,
    out_type=(
        jax.ShapeDtypeStruct((H0,), f32),
        jax.ShapeDtypeStruct((H0, D), f32),
        jax.ShapeDtypeStruct((N,), f32),
    ),
    mesh=_mesh,
    compiler_params=pltpu.CompilerParams(needs_layout_passes=False),
    scratch_types=[
        pltpu.VMEM_SHARED((N,), f32),
        pltpu.VMEM((N,), f32),
        pltpu.VMEM((CB,), f32),
        pltpu.VMEM((CB,), i32),
        pltpu.VMEM((CB,), i32),
        pltpu.VMEM((NSUB, SUB), i32),
        pltpu.VMEM((NSUB, SUB), i32),
        pltpu.VMEM((CB, D), f32),
        pltpu.VMEM((CB,), f32),
        pltpu.SemaphoreType.DMA,
    ],
)
def _sc_seg_a(w0_hbm, w1_hbm, src_hbm, dst_hbm, tab_hbm, zeros_hbm,
              a0_out, r0_out, s_out,
              S_sh, stile, wv, srcbig, dstbig, srcv, dstv, rows, sbuf, sem):
    cid = lax.axis_index("c")
    sid = lax.axis_index("s")
    wid = _wid()

    @pl.when(sid == 0)
    def _():
        pltpu.sync_copy(zeros_hbm, S_sh)

    plsc.subcore_barrier()

    def ph1a(i, carry):
        lbase = sid * (H0 // NS) + i * CB
        pltpu.sync_copy(w0_hbm.at[pl.ds(lbase, CB)], wv)
        pltpu.sync_copy(src_hbm.at[pl.ds(lbase, CB)], srcbig)
        _stage_idx(srcbig, srcv)
        descs = []
        for j in range(NSUB):
            descs.append(pltpu.async_copy(
                wv.at[pl.ds(j * SUB, SUB)], S_sh.at[srcv.at[j]], sem,
                add=True))
        for d in descs:
            d.wait()
        return carry

    def ph1b(i, carry):
        lbase = sid * (H1 // NS) + i * CB
        pltpu.sync_copy(w1_hbm.at[pl.ds(lbase, CB)], wv)
        pltpu.sync_copy(src_hbm.at[pl.ds(H0 + lbase, CB)], srcbig)
        _stage_idx(srcbig, srcv)
        descs = []
        for j in range(NSUB):
            descs.append(pltpu.async_copy(
                wv.at[pl.ds(j * SUB, SUB)], S_sh.at[srcv.at[j]], sem,
                add=True))
        for d in descs:
            d.wait()
        return carry

    lax.fori_loop(0, H0 // NS // CB, ph1a, 0)
    lax.fori_loop(0, H1 // NS // CB, ph1b, 0)
    plsc.subcore_barrier()
    pltpu.sync_copy(S_sh, stile)

    @pl.when((sid == 0) & (cid == 0))
    def _():
        pltpu.sync_copy(S_sh, s_out)

    def ph2(i, carry):
        base = wid * EW0 + i * CB
        pltpu.sync_copy(src_hbm.at[pl.ds(base, CB)], srcbig)
        pltpu.sync_copy(dst_hbm.at[pl.ds(base, CB)], dstbig)
        pltpu.sync_copy(w0_hbm.at[pl.ds(base, CB)], wv)
        _stage_idx(dstbig, dstv)
        descs = []
        for j in range(NSUB):
            descs.append(pltpu.async_copy(
                tab_hbm.at[dstv.at[j]], rows.at[pl.ds(j * SUB, SUB)], sem))
        for k in range(CB // 16):
            idx = srcbig[pl.ds(k * 16, 16)]
            sv = plsc.load_gather(stile, [idx])
            sbuf[pl.ds(k * 16, 16)] = wv[pl.ds(k * 16, 16)] / (sv + 1e-16)
        for d in descs:
            d.wait()
        pltpu.sync_copy(rows, r0_out.at[pl.ds(base, CB)])
        pltpu.sync_copy(sbuf, a0_out.at[pl.ds(base, CB)])
        return carry

    lax.fori_loop(0, EW0 // CB, ph2, 0)


@functools.partial(
    pl.kernel,
    out_type=(
        jax.ShapeDtypeStruct((H1,), f32),
        jax.ShapeDtypeStruct((H1, D), f32),
    ),
    mesh=_mesh,
    compiler_params=pltpu.CompilerParams(needs_layout_passes=False),
    scratch_types=[
        pltpu.VMEM((N,), f32),
        pltpu.VMEM((CB,), f32),
        pltpu.VMEM((CB,), i32),
        pltpu.VMEM((CB,), i32),
        pltpu.VMEM((NSUB, SUB), i32),
        pltpu.VMEM((CB, D), f32),
        pltpu.VMEM((CB,), f32),
        pltpu.SemaphoreType.DMA,
    ],
)
def _sc_seg_b(s_hbm, w1_hbm, src_hbm, dst_hbm, tab_hbm,
              a1_out, r1_out,
              stile, wv, srcbig, dstbig, dstv, rows, sbuf, sem):
    wid = _wid()
    pltpu.sync_copy(s_hbm, stile)

    def ph2(i, carry):
        lbase = wid * EW1 + i * CB
        gbase = H0 + lbase
        pltpu.sync_copy(src_hbm.at[pl.ds(gbase, CB)], srcbig)
        pltpu.sync_copy(dst_hbm.at[pl.ds(gbase, CB)], dstbig)
        pltpu.sync_copy(w1_hbm.at[pl.ds(lbase, CB)], wv)
        _stage_idx(dstbig, dstv)
        descs = []
        for j in range(NSUB):
            descs.append(pltpu.async_copy(
                tab_hbm.at[dstv.at[j]], rows.at[pl.ds(j * SUB, SUB)], sem))
        for k in range(CB // 16):
            idx = srcbig[pl.ds(k * 16, 16)]
            sv = plsc.load_gather(stile, [idx])
            sbuf[pl.ds(k * 16, 16)] = wv[pl.ds(k * 16, 16)] / (sv + 1e-16)
        for d in descs:
            d.wait()
        pltpu.sync_copy(rows, r1_out.at[pl.ds(lbase, CB)])
        pltpu.sync_copy(sbuf, a1_out.at[pl.ds(lbase, CB)])
        return carry

    lax.fori_loop(0, EW1 // CB, ph2, 0)


NH = N // 2
NHP = NH + 8
CBS = 160
NSS = CBS // SUB


def _make_scatter(eoff, esz):
    nits = esz // NS // CBS
    tsz = esz // NS

    @functools.partial(
        pl.kernel,
        out_type=jax.ShapeDtypeStruct((N, D), f32),
        mesh=_mesh,
        compiler_params=pltpu.CompilerParams(needs_layout_passes=False),
        scratch_types=[
            pltpu.VMEM_SHARED((NHP, D), f32),
            pltpu.VMEM((CBS, D), f32),
            pltpu.VMEM((CBS, D), f32),
            pltpu.VMEM((CBS,), i32),
            pltpu.VMEM((CBS,), i32),
            pltpu.VMEM((NSS, SUB), i32),
            pltpu.VMEM((NSS, SUB), i32),
            pltpu.SemaphoreType.DMA,
            pltpu.SemaphoreType.DMA,
            pltpu.SemaphoreType.DMA,
            pltpu.SemaphoreType.DMA,
        ],
    )
    def k(m_hbm, dst_hbm, zeros_hbm, out,
          X_sh, rowsA, rowsB, dstbigA, dstbigB, dstvA, dstvB,
          lsemA, lsemB, ssemA, ssemB):
        cid = lax.axis_index("c")
        sid = lax.axis_index("s")
        lo = cid * NH
        bufs = ((rowsA, dstbigA, dstvA, lsemA, ssemA),
                (rowsB, dstbigB, dstvB, lsemB, ssemB))

        @pl.when(sid == 0)
        def _():
            pltpu.sync_copy(zeros_hbm.at[pl.ds(0, NHP)], X_sh)

        plsc.subcore_barrier()

        base0 = sid * tsz
        pltpu.async_copy(m_hbm.at[pl.ds(base0, CBS)], rowsA, lsemA)
        pltpu.async_copy(dst_hbm.at[pl.ds(eoff + base0, CBS)], dstbigA, lsemA)

        def chunk(c, b_):
            rows, dstbig, dstv, lsem, ssem = bufs[b_]
            rowsN, dstbigN, dstvN, lsemN, ssemN = bufs[1 - b_]

            @pl.when(c >= 1)
            def _():
                for j in range(NSS):
                    pltpu.make_async_copy(
                        rowsN.at[pl.ds(j * SUB, SUB)],
                        X_sh.at[dstvN.at[j]], ssemN).wait()

            @pl.when(c + 1 < nits)
            def _():
                nbase = sid * tsz + (c + 1) * CBS
                pltpu.async_copy(m_hbm.at[pl.ds(nbase, CBS)], rowsN, lsemN)
                pltpu.async_copy(dst_hbm.at[pl.ds(eoff + nbase, CBS)],
                                 dstbigN, lsemN)

            pltpu.make_async_copy(m_hbm.at[pl.ds(0, CBS)], rows, lsem).wait()
            pltpu.make_async_copy(dst_hbm.at[pl.ds(0, CBS)], dstbig,
                                  lsem).wait()
            for kk in range(CBS // 16):
                t = dstbig[pl.ds(kk * 16, 16)] - lo
                t = jnp.where((t >= 0) & (t < NH), t, NH)
                dstv[kk // (SUB // 16),
                     pl.ds((kk % (SUB // 16)) * 16, 16)] = t
            for j in range(NSS):
                pltpu.async_copy(rows.at[pl.ds(j * SUB, SUB)],
                                 X_sh.at[dstv.at[j]], ssem, add=True)

        def body(c, carry):
            for b_ in (0, 1):
                @pl.when(lax.rem(c, 2) == b_)
                def _(b_=b_):
                    chunk(c, b_)
            return carry

        lax.fori_loop(0, nits, body, 0)
        rowsL, dstbigL, dstvL, lsemL, ssemL = bufs[(nits - 1) % 2]
        for j in range(NSS):
            pltpu.make_async_copy(rowsL.at[pl.ds(j * SUB, SUB)],
                                  X_sh.at[dstvL.at[j]], ssemL).wait()
        plsc.subcore_barrier()

        @pl.when(sid == 0)
        def _():
            pltpu.sync_copy(X_sh.at[pl.ds(0, NH)], out.at[pl.ds(lo, NH)])

    return k


_sc_scatter_h = (_make_scatter(0, H0), _make_scatter(H0, H1))


@functools.partial(
    pl.kernel,
    out_type=(
        jax.ShapeDtypeStruct((H0,), f32),
        jax.ShapeDtypeStruct((H0, D), f32),
        jax.ShapeDtypeStruct((N,), f32),
    ),
    mesh=_mesh,
    compiler_params=pltpu.CompilerParams(needs_layout_passes=False),
    scratch_types=[
        pltpu.VMEM_SHARED((N,), f32),
        pltpu.VMEM((N,), f32),
        pltpu.VMEM((N,), f32),
        pltpu.VMEM((N,), f32),
        pltpu.VMEM((CB,), f32),
        pltpu.VMEM((CB,), i32),
        pltpu.VMEM((CB,), i32),
        pltpu.VMEM((NSUB, SUB), i32),
        pltpu.VMEM((NSUB, SUB), i32),
        pltpu.VMEM((CB, D), f32),
        pltpu.VMEM((CB,), f32),
        pltpu.SemaphoreType.DMA,
    ],
)
def _sc_e1_a(p1_hbm, p2_hbm, src_hbm, dst_hbm, x1_hbm, zeros_hbm,
             a0_out, r0_out, s_out,
             S_sh, p1t, p2t, stile, wv, srcbig, dstbig, srcv, dstv, rows,
             sbuf, sem):
    cid = lax.axis_index("c")
    sid = lax.axis_index("s")
    wid = _wid()
    pltpu.sync_copy(p1_hbm, p1t)
    pltpu.sync_copy(p2_hbm, p2t)

    @pl.when(sid == 0)
    def _():
        pltpu.sync_copy(zeros_hbm, S_sh)

    plsc.subcore_barrier()

    def ph1(i, carry):
        base = sid * ET + i * CB
        pltpu.sync_copy(src_hbm.at[pl.ds(base, CB)], srcbig)
        pltpu.sync_copy(dst_hbm.at[pl.ds(base, CB)], dstbig)
        _stage_idx(srcbig, srcv)
        for k in range(CB // 16):
            idxd = dstbig[pl.ds(k * 16, 16)]
            idxs = srcbig[pl.ds(k * 16, 16)]
            t = plsc.load_gather(p1t, [idxd]) + plsc.load_gather(p2t, [idxs])
            t = jnp.where(t >= 0.0, t, 0.01 * t)
            wv[pl.ds(k * 16, 16)] = jnp.exp(t)
        descs = []
        for j in range(NSUB):
            descs.append(pltpu.async_copy(
                wv.at[pl.ds(j * SUB, SUB)], S_sh.at[srcv.at[j]], sem,
                add=True))
        for d in descs:
            d.wait()
        return carry

    lax.fori_loop(0, ET // CB, ph1, 0)
    plsc.subcore_barrier()
    pltpu.sync_copy(S_sh, stile)

    @pl.when((sid == 0) & (cid == 0))
    def _():
        pltpu.sync_copy(S_sh, s_out)

    def ph2(i, carry):
        base = wid * EW0 + i * CB
        pltpu.sync_copy(src_hbm.at[pl.ds(base, CB)], srcbig)
        pltpu.sync_copy(dst_hbm.at[pl.ds(base, CB)], dstbig)
        _stage_idx(dstbig, dstv)
        descs = []
        for j in range(NSUB):
            descs.append(pltpu.async_copy(
                x1_hbm.at[dstv.at[j]], rows.at[pl.ds(j * SUB, SUB)], sem))
        for k in range(CB // 16):
            idxd = dstbig[pl.ds(k * 16, 16)]
            idxs = srcbig[pl.ds(k * 16, 16)]
            t = plsc.load_gather(p1t, [idxd]) + plsc.load_gather(p2t, [idxs])
            t = jnp.where(t >= 0.0, t, 0.01 * t)
            sv = plsc.load_gather(stile, [idxs])
            sbuf[pl.ds(k * 16, 16)] = jnp.exp(t) / (sv + 1e-16)
        for d in descs:
            d.wait()
        pltpu.sync_copy(rows, r0_out.at[pl.ds(base, CB)])
        pltpu.sync_copy(sbuf, a0_out.at[pl.ds(base, CB)])
        return carry

    lax.fori_loop(0, EW0 // CB, ph2, 0)


@functools.partial(
    pl.kernel,
    out_type=(
        jax.ShapeDtypeStruct((H1,), f32),
        jax.ShapeDtypeStruct((H1, D), f32),
    ),
    mesh=_mesh,
    compiler_params=pltpu.CompilerParams(needs_layout_passes=False),
    scratch_types=[
        pltpu.VMEM((N,), f32),
        pltpu.VMEM((N,), f32),
        pltpu.VMEM((N,), f32),
        pltpu.VMEM((CB,), i32),
        pltpu.VMEM((CB,), i32),
        pltpu.VMEM((NSUB, SUB), i32),
        pltpu.VMEM((CB, D), f32),
        pltpu.VMEM((CB,), f32),
        pltpu.SemaphoreType.DMA,
    ],
)
def _sc_e1_b(p1_hbm, p2_hbm, s_hbm, src_hbm, dst_hbm, x1_hbm,
             a1_out, r1_out,
             p1t, p2t, stile, srcbig, dstbig, dstv, rows, sbuf, sem):
    wid = _wid()
    pltpu.sync_copy(p1_hbm, p1t)
    pltpu.sync_copy(p2_hbm, p2t)
    pltpu.sync_copy(s_hbm, stile)

    def ph2(i, carry):
        lbase = wid * EW1 + i * CB
        gbase = H0 + lbase
        pltpu.sync_copy(src_hbm.at[pl.ds(gbase, CB)], srcbig)
        pltpu.sync_copy(dst_hbm.at[pl.ds(gbase, CB)], dstbig)
        _stage_idx(dstbig, dstv)
        descs = []
        for j in range(NSUB):
            descs.append(pltpu.async_copy(
                x1_hbm.at[dstv.at[j]], rows.at[pl.ds(j * SUB, SUB)], sem))
        for k in range(CB // 16):
            idxd = dstbig[pl.ds(k * 16, 16)]
            idxs = srcbig[pl.ds(k * 16, 16)]
            t = plsc.load_gather(p1t, [idxd]) + plsc.load_gather(p2t, [idxs])
            t = jnp.where(t >= 0.0, t, 0.01 * t)
            sv = plsc.load_gather(stile, [idxs])
            sbuf[pl.ds(k * 16, 16)] = jnp.exp(t) / (sv + 1e-16)
        for d in descs:
            d.wait()
        pltpu.sync_copy(rows, r1_out.at[pl.ds(lbase, CB)])
        pltpu.sync_copy(sbuf, a1_out.at[pl.ds(lbase, CB)])
        return carry

    lax.fori_loop(0, EW1 // CB, ph2, 0)


def _leaky(t):
    return jnp.where(t >= 0.0, t, 0.01 * t)


def _dot(a, b):
    return jnp.dot(a, b, preferred_element_type=f32)


def _tc_prep0_body(x_ref, wa_ref, ba_ref, wn_ref, wal_ref, u_ref, v_ref, p_ref):
    x = x_ref[...]
    u = _leaky(_dot(x, wa_ref[...]) + ba_ref[...])
    u_ref[...] = u
    v_ref[...] = _dot(x, wn_ref[...])
    p_ref[...] = _dot(u, wal_ref[...])


def _tc_prep0(x, W_atom0, b_atom0, Wnb1, Wal0a):
    return pl.pallas_call(
        _tc_prep0_body,
        out_shape=(
            jax.ShapeDtypeStruct((N, D), f32),
            jax.ShapeDtypeStruct((N, D), f32),
            jax.ShapeDtypeStruct((N, 1), f32),
        ),
    )(x, W_atom0, b_atom0.reshape(1, D), Wnb1, Wal0a)


BE = 3200
BEH = BE // 128


def _tc_edge0a_body(vs_ref, ea_ref, pd_ref, wn2_ref, bn_ref, wb_ref, bal_ref,
                    w_ref):
    ea = lax.dot_general(ea_ref[...], wn2_ref[...], (((0,), (0,)), ((), ())),
                         preferred_element_type=f32)
    xj = _leaky(vs_ref[...] + ea + bn_ref[...])
    ej = _dot(xj, wb_ref[...])
    pdt = jnp.transpose(pd_ref[0])
    pdcol = jnp.concatenate([pdt[:, r:r + 1] for r in range(BEH)], axis=0)
    e = _leaky(pdcol + ej + bal_ref[...])
    w = jnp.exp(e)
    rows = [jnp.transpose(w[r * 128:(r + 1) * 128, :]) for r in range(BEH)]
    w_ref[...] = jnp.reshape(jnp.concatenate(rows, axis=0), (1, BEH, 128))


def _make_tc_edge0a(eoff, esz):
    def run(Vsrc, eaT, Pdst, Wnb2, b_nb0, Wb0, b_align0):
        full = lambda sh: pl.BlockSpec(sh, lambda i: (0, 0))
        boff = eoff // BE
        return pl.pallas_call(
            _tc_edge0a_body,
            grid=(esz // BE,),
            compiler_params=pltpu.CompilerParams(
                fuse_transposed_lhs_in_matmul=True),
            in_specs=[
                pl.BlockSpec((BE, D), lambda i: (i, 0)),
                pl.BlockSpec((16, BE), lambda i: (0, i + boff)),
                pl.BlockSpec((1, BEH, 128), lambda i: (i, 0, 0)),
                full((16, D)),
                full((1, D)),
                full((D, 1)),
                full((1, 1)),
            ],
            out_specs=pl.BlockSpec((1, BEH, 128), lambda i: (i, 0, 0)),
            out_shape=jax.ShapeDtypeStruct((esz // BE, BEH, 128), f32),
        )(Vsrc, eaT, Pdst, Wnb2, b_nb0.reshape(1, D), Wb0,
          b_align0.reshape(1, 1))
    return run


_tc_edge0a_h = (_make_tc_edge0a(0, H0), _make_tc_edge0a(H0, H1))


def _tc_edge_msg_body(xi_ref, a_ref, watt_ref, batt_ref,
                      wih_ref, bih_ref, whh_ref, bhh_ref, m_ref):
    x_i = xi_ref[...]
    at = jnp.transpose(a_ref[0])
    q = _dot(x_i, watt_ref[...]) + batt_ref[...]
    aq = jnp.concatenate(
        [q[r * 128:(r + 1) * 128, :] * at[:, r:r + 1] for r in range(BEH)],
        axis=0)
    c = jnp.where(aq > 0.0, aq, jnp.exp(aq) - 1.0)
    gi = _dot(c, wih_ref[...]) + bih_ref[...]
    gh = _dot(x_i, whh_ref[...]) + bhh_ref[...]
    r = jax.nn.sigmoid(gi[:, :D] + gh[:, :D])
    z = jax.nn.sigmoid(gi[:, D:2 * D] + gh[:, D:2 * D])
    n = jnp.tanh(gi[:, 2 * D:] + r * gh[:, 2 * D:])
    m_ref[...] = (1.0 - z) * n + z * x_i


def _make_tc_edge_msg(esz):
    def run(Xd, a, W_att, b_att, WihT, bih, WhhT, bhh):
        full = lambda sh: pl.BlockSpec(sh, lambda i: (0, 0))
        return pl.pallas_call(
            _tc_edge_msg_body,
            grid=(esz // BE,),
            in_specs=[
                pl.BlockSpec((BE, D), lambda i: (i, 0)),
                pl.BlockSpec((1, BEH, 128), lambda i: (i, 0, 0)),
                full((D, D)),
                full((1, D)),
                full((D, 3 * D)),
                full((1, 3 * D)),
                full((D, 3 * D)),
                full((1, 3 * D)),
            ],
            out_specs=pl.BlockSpec((BE, D), lambda i: (i, 0)),
            out_shape=jax.ShapeDtypeStruct((esz, D), f32),
        )(Xd, a, W_att, b_att.reshape(1, D), WihT, bih.reshape(1, 3 * D),
          WhhT, bhh.reshape(1, 3 * D))
    return run


_tc_edge_msg_h = (_make_tc_edge_msg(H0), _make_tc_edge_msg(H1))


def _tc_prep1_body(xa_ref, xb_ref, wa_ref, wb_ref, bal_ref,
                   x1_ref, p1_ref, p2_ref):
    x1 = xa_ref[...] + xb_ref[...]
    x1_ref[...] = x1
    p1_ref[...] = _dot(x1, wa_ref[...]) + bal_ref[...]
    p2_ref[...] = _dot(x1, wb_ref[...])


def _tc_prep1(xa, xb, Wa1, Wb1, b_align1):
    return pl.pallas_call(
        _tc_prep1_body,
        out_shape=(
            jax.ShapeDtypeStruct((N, D), f32),
            jax.ShapeDtypeStruct((N, 1), f32),
            jax.ShapeDtypeStruct((N, 1), f32),
        ),
    )(xa, xb, Wa1, Wb1, b_align1.reshape(1, 1))


def _tc_final_body(xa_ref, xb_ref, out_ref):
    out_ref[...] = xa_ref[...] + xb_ref[...]


def _tc_final(xa, xb):
    return pl.pallas_call(
        _tc_final_body,
        out_shape=jax.ShapeDtypeStruct((N, D), f32),
    )(xa, xb)


def kernel(x, edge_index, edge_attr,
           W_atom0, b_atom0, W_nb0, b_nb0, W_align0, b_align0, W_att0, b_att0,
           Wih0, Whh0, bih0, bhh0,
           W_align1, b_align1, W_att1, b_att1, Wih1, Whh1, bih1, bhh1):
    src = edge_index[0]
    dst = edge_index[1]
    zeros_n = jnp.zeros((N,), f32)
    zeros_nd = jnp.zeros((N, D), f32)

    u, v, p = _tc_prep0(x, W_atom0, b_atom0, W_nb0[:D], W_align0[:D])
    pflat = p.reshape(N)
    eaT = edge_attr.T
    Vs0, Pd0 = _sc_gather0_h[0](v, pflat, src, dst)
    Vs1, Pd1 = _sc_gather0_h[1](v, pflat, src, dst)
    w0 = _tc_edge0a_h[0](Vs0, eaT, Pd0.reshape(H0 // BE, BEH, 128),
                         W_nb0[D:], b_nb0, W_align0[D:], b_align0)
    w1 = _tc_edge0a_h[1](Vs1, eaT, Pd1.reshape(H1 // BE, BEH, 128),
                         W_nb0[D:], b_nb0, W_align0[D:], b_align0)
    w0f = w0.reshape(H0)
    w1f = w1.reshape(H1)
    A0, Ud0, S0 = _sc_seg_a(w0f, w1f, src, dst, u, zeros_n)
    A1, Ud1 = _sc_seg_b(S0, w1f, src, dst, u)
    M0 = _tc_edge_msg_h[0](Ud0, A0.reshape(H0 // BE, BEH, 128),
                           W_att0, b_att0, Wih0.T, bih0, Whh0.T, bhh0)
    M1 = _tc_edge_msg_h[1](Ud1, A1.reshape(H1 // BE, BEH, 128),
                           W_att0, b_att0, Wih0.T, bih0, Whh0.T, bhh0)
    xp0 = _sc_scatter_h[0](M0, dst, zeros_nd)
    xp1 = _sc_scatter_h[1](M1, dst, zeros_nd)

    x1, p1, p2 = _tc_prep1(xp0, xp1, W_align1[:D], W_align1[D:], b_align1)
    p1f = p1.reshape(N)
    p2f = p2.reshape(N)
    B0, Xd0, S1 = _sc_e1_a(p1f, p2f, src, dst, x1, zeros_n)
    B1, Xd1 = _sc_e1_b(p1f, p2f, S1, src, dst, x1)
    N0 = _tc_edge_msg_h[0](Xd0, B0.reshape(H0 // BE, BEH, 128),
                           W_att1, b_att1, Wih1.T, bih1, Whh1.T, bhh1)
    N1 = _tc_edge_msg_h[1](Xd1, B1.reshape(H1 // BE, BEH, 128),
                           W_att1, b_att1, Wih1.T, bih1, Whh1.T, bhh1)
    yp0 = _sc_scatter_h[0](N0, dst, zeros_nd)
    yp1 = _sc_scatter_h[1](N1, dst, zeros_nd)
    return _tc_final(yp0, yp1)
